# Initial kernel scaffold; baseline (speedup 1.0000x reference)
import jax
import jax.numpy as jnp
from jax.experimental import pallas as pl


def kernel(x_var, x_con, x_soc, v2c_src, v2c_dst, s2c_src, s2c_dst, Wv, bv, Wc, bc, Ws, bs, W1f, b1f, W1b, b1b, W2f, b2f, W2b, b2b, Wo1, bo1, Wo2, bo2, Wo3, bo3):
    raise NotImplementedError("write your pallas kernel here")


# speedup vs baseline: 31.8460x; 31.8460x over previous
import functools

import jax
import jax.numpy as jnp
from jax import lax
from jax.experimental import pallas as pl
from jax.experimental.pallas import tpu as pltpu
from jax.experimental.pallas import tpu_sc as plsc

N_VAR, N_CON, N_SOC = 100000, 50000, 10000
E_VC, E_SC = 3200000, 160000
H = 16
NC, NS = 2, 16
NW = NC * NS


def _padded(n):
    per = -(-n // NS)
    per = -(-per // 8) * 8
    return per * NS, per


NVP, VPER = _padded(N_VAR)
NCP, CPER = _padded(N_CON)
NSP, SPER = _padded(N_SOC)

EVW = E_VC // NW
ESW = E_SC // NW
KV = 2000
KB = 1000
KS = 1000

_MESH = plsc.VectorSubcoreMesh(core_axis_name="c", subcore_axis_name="s")
_SC_PARAMS = pltpu.CompilerParams(use_tc_tiling_on_sc=False)
_f32 = jnp.float32


@functools.partial(
    pl.kernel,
    out_type=(jax.ShapeDtypeStruct((NC * NVP,), _f32),
              jax.ShapeDtypeStruct((NC * NSP,), _f32)),
    mesh=_MESH,
    compiler_params=_SC_PARAMS,
    scratch_types=[
        pltpu.VMEM_SHARED((NVP,), _f32),
        pltpu.VMEM_SHARED((NSP,), _f32),
        pltpu.VMEM((KV,), jnp.int32),
        pltpu.VMEM((KS,), jnp.int32),
        pltpu.VMEM((KV,), _f32),
        pltpu.VMEM((KS,), _f32),
        pltpu.VMEM((VPER,), _f32),
    ],
)
def _sc_deg_src(vsrc_hbm, ssrc_hbm, zeros1_hbm, ones_hbm,
                dv_out, ds_out, dv_acc, ds_acc, idxv, idxs, onev, ones, stg1):
    c = lax.axis_index("c")
    s = lax.axis_index("s")
    wid = c * NS + s
    pltpu.sync_copy(zeros1_hbm.at[pl.ds(0, VPER)], stg1)
    pltpu.sync_copy(stg1, dv_acc.at[pl.ds(s * VPER, VPER)])
    pltpu.sync_copy(stg1.at[pl.ds(0, SPER)], ds_acc.at[pl.ds(s * SPER, SPER)])
    pltpu.sync_copy(ones_hbm, onev)
    pltpu.sync_copy(ones_hbm.at[pl.ds(0, KS)], ones)
    plsc.subcore_barrier()

    def bodyv(i, carry):
        base = pl.multiple_of(wid * EVW + i * KV, 8)
        pltpu.sync_copy(vsrc_hbm.at[pl.ds(base, KV)], idxv)
        pltpu.sync_copy(onev, dv_acc.at[idxv], add=True)
        return carry

    lax.fori_loop(0, EVW // KV, bodyv, 0)

    def bodys(i, carry):
        base = pl.multiple_of(wid * ESW + i * KS, 8)
        pltpu.sync_copy(ssrc_hbm.at[pl.ds(base, KS)], idxs)
        pltpu.sync_copy(ones, ds_acc.at[idxs], add=True)
        return carry

    lax.fori_loop(0, ESW // KS, bodys, 0)
    plsc.subcore_barrier()
    pltpu.sync_copy(dv_acc.at[pl.ds(s * VPER, VPER)], stg1)
    pltpu.sync_copy(stg1, dv_out.at[pl.ds(c * NVP + s * VPER, VPER)])
    pltpu.sync_copy(ds_acc.at[pl.ds(s * SPER, SPER)], stg1.at[pl.ds(0, SPER)])
    pltpu.sync_copy(stg1.at[pl.ds(0, SPER)],
                    ds_out.at[pl.ds(c * NSP + s * SPER, SPER)])


@functools.partial(
    pl.kernel,
    out_type=(jax.ShapeDtypeStruct((NC * NCP, H), _f32),
              jax.ShapeDtypeStruct((NC * NCP,), _f32)),
    mesh=_MESH,
    compiler_params=_SC_PARAMS,
    scratch_types=[
        pltpu.VMEM_SHARED((NCP, H), _f32),
        pltpu.VMEM_SHARED((NCP,), _f32),
        pltpu.VMEM((KV,), jnp.int32),
        pltpu.VMEM((KV,), jnp.int32),
        pltpu.VMEM((KV, H), _f32),
        pltpu.VMEM((KV,), _f32),
        pltpu.VMEM((CPER,), _f32),
        pltpu.SemaphoreType.DMA,
    ],
)
def _sc_fwd_v(src_hbm, dst_hbm, tab_hbm, zeros2_hbm, zeros1_hbm, ones_hbm,
              agg_out, deg_out, agg_acc, deg_acc,
              sidx, didx, rows, onev, stg1, sem):
    c = lax.axis_index("c")
    s = lax.axis_index("s")
    wid = c * NS + s
    pltpu.sync_copy(zeros2_hbm.at[pl.ds(0, KV)], rows)
    pltpu.sync_copy(rows, agg_acc.at[pl.ds(s * CPER, KV)])
    pltpu.sync_copy(rows.at[pl.ds(0, CPER - KV)],
                    agg_acc.at[pl.ds(s * CPER + KV, CPER - KV)])
    pltpu.sync_copy(zeros1_hbm.at[pl.ds(0, CPER)], stg1)
    pltpu.sync_copy(stg1, deg_acc.at[pl.ds(s * CPER, CPER)])
    pltpu.sync_copy(ones_hbm, onev)
    plsc.subcore_barrier()

    def body(i, carry):
        base = pl.multiple_of(wid * EVW + i * KV, 8)
        pltpu.sync_copy(src_hbm.at[pl.ds(base, KV)], sidx)
        pltpu.sync_copy(dst_hbm.at[pl.ds(base, KV)], didx)
        pltpu.async_copy(tab_hbm.at[sidx], rows, sem).wait()
        pltpu.sync_copy(rows, agg_acc.at[didx], add=True)
        pltpu.sync_copy(onev, deg_acc.at[didx], add=True)
        return carry

    lax.fori_loop(0, EVW // KV, body, 0)
    plsc.subcore_barrier()
    pltpu.sync_copy(agg_acc.at[pl.ds(s * CPER, KV)], rows)
    pltpu.sync_copy(rows, agg_out.at[pl.ds(c * NCP + s * CPER, KV)])
    pltpu.sync_copy(agg_acc.at[pl.ds(s * CPER + KV, CPER - KV)],
                    rows.at[pl.ds(0, CPER - KV)])
    pltpu.sync_copy(rows.at[pl.ds(0, CPER - KV)],
                    agg_out.at[pl.ds(c * NCP + s * CPER + KV, CPER - KV)])
    pltpu.sync_copy(deg_acc.at[pl.ds(s * CPER, CPER)], stg1)
    pltpu.sync_copy(stg1, deg_out.at[pl.ds(c * NCP + s * CPER, CPER)])


@functools.partial(
    pl.kernel,
    out_type=(jax.ShapeDtypeStruct((NC * NCP, H), _f32),
              jax.ShapeDtypeStruct((NC * NCP,), _f32)),
    mesh=_MESH,
    compiler_params=_SC_PARAMS,
    scratch_types=[
        pltpu.VMEM_SHARED((NCP, H), _f32),
        pltpu.VMEM_SHARED((NCP,), _f32),
        pltpu.VMEM((KS,), jnp.int32),
        pltpu.VMEM((KS,), jnp.int32),
        pltpu.VMEM((KS, H), _f32),
        pltpu.VMEM((KS,), _f32),
        pltpu.VMEM((CPER,), _f32),
        pltpu.SemaphoreType.DMA,
    ],
)
def _sc_fwd_s(src_hbm, dst_hbm, tab_hbm, zeros2_hbm, zeros1_hbm, ones_hbm,
              agg_out, deg_out, agg_acc, deg_acc,
              sidx, didx, rows, onev, stg1, sem):
    c = lax.axis_index("c")
    s = lax.axis_index("s")
    wid = c * NS + s
    nz = CPER // KS
    rz = CPER - nz * KS
    pltpu.sync_copy(zeros2_hbm.at[pl.ds(0, KS)], rows)

    def zbody(k, carry):
        off = pl.multiple_of(s * CPER + k * KS, 8)
        pltpu.sync_copy(rows, agg_acc.at[pl.ds(off, KS)])
        return carry

    lax.fori_loop(0, nz, zbody, 0)
    pltpu.sync_copy(rows.at[pl.ds(0, rz)],
                    agg_acc.at[pl.ds(s * CPER + nz * KS, rz)])
    pltpu.sync_copy(zeros1_hbm.at[pl.ds(0, CPER)], stg1)
    pltpu.sync_copy(stg1, deg_acc.at[pl.ds(s * CPER, CPER)])
    pltpu.sync_copy(ones_hbm.at[pl.ds(0, KS)], onev)
    plsc.subcore_barrier()

    def body(i, carry):
        base = pl.multiple_of(wid * ESW + i * KS, 8)
        pltpu.sync_copy(src_hbm.at[pl.ds(base, KS)], sidx)
        pltpu.sync_copy(dst_hbm.at[pl.ds(base, KS)], didx)
        pltpu.async_copy(tab_hbm.at[sidx], rows, sem).wait()
        pltpu.sync_copy(rows, agg_acc.at[didx], add=True)
        pltpu.sync_copy(onev, deg_acc.at[didx], add=True)
        return carry

    lax.fori_loop(0, ESW // KS, body, 0)
    plsc.subcore_barrier()

    def obody(k, carry):
        off = pl.multiple_of(s * CPER + k * KS, 8)
        off2 = pl.multiple_of(c * NCP + s * CPER + k * KS, 8)
        pltpu.sync_copy(agg_acc.at[pl.ds(off, KS)], rows)
        pltpu.sync_copy(rows, agg_out.at[pl.ds(off2, KS)])
        return carry

    lax.fori_loop(0, nz, obody, 0)
    pltpu.sync_copy(agg_acc.at[pl.ds(s * CPER + nz * KS, rz)],
                    rows.at[pl.ds(0, rz)])
    pltpu.sync_copy(rows.at[pl.ds(0, rz)],
                    agg_out.at[pl.ds(c * NCP + s * CPER + nz * KS, rz)])
    pltpu.sync_copy(deg_acc.at[pl.ds(s * CPER, CPER)], stg1)
    pltpu.sync_copy(stg1, deg_out.at[pl.ds(c * NCP + s * CPER, CPER)])


@functools.partial(
    pl.kernel,
    out_type=jax.ShapeDtypeStruct((NC * NVP, H), _f32),
    mesh=_MESH,
    compiler_params=_SC_PARAMS,
    scratch_types=[
        pltpu.VMEM_SHARED((NVP, H), _f32),
        pltpu.VMEM((KB,), jnp.int32),
        pltpu.VMEM((KB,), jnp.int32),
        pltpu.VMEM((KB, H), _f32),
        pltpu.SemaphoreType.DMA,
    ],
)
def _sc_bwd(gidx_hbm, sidx_hbm, tab_hbm, zeros2_hbm,
            u_out, u_acc, gidx, sidx, rows, sem):
    c = lax.axis_index("c")
    s = lax.axis_index("s")
    wid = c * NS + s
    nz = VPER // KB
    rz = VPER - nz * KB
    pltpu.sync_copy(zeros2_hbm.at[pl.ds(0, KB)], rows)

    def zbody(k, carry):
        off = pl.multiple_of(s * VPER + k * KB, 8)
        pltpu.sync_copy(rows, u_acc.at[pl.ds(off, KB)])
        return carry

    lax.fori_loop(0, nz, zbody, 0)
    pltpu.sync_copy(rows.at[pl.ds(0, rz)],
                    u_acc.at[pl.ds(s * VPER + nz * KB, rz)])
    plsc.subcore_barrier()

    def body(i, carry):
        base = pl.multiple_of(wid * EVW + i * KB, 8)
        pltpu.sync_copy(gidx_hbm.at[pl.ds(base, KB)], gidx)
        pltpu.sync_copy(sidx_hbm.at[pl.ds(base, KB)], sidx)
        pltpu.async_copy(tab_hbm.at[gidx], rows, sem).wait()
        pltpu.sync_copy(rows, u_acc.at[sidx], add=True)
        return carry

    lax.fori_loop(0, EVW // KB, body, 0)
    plsc.subcore_barrier()

    def obody(k, carry):
        off = pl.multiple_of(s * VPER + k * KB, 8)
        off2 = pl.multiple_of(c * NVP + s * VPER + k * KB, 8)
        pltpu.sync_copy(u_acc.at[pl.ds(off, KB)], rows)
        pltpu.sync_copy(rows, u_out.at[pl.ds(off2, KB)])
        return carry

    lax.fori_loop(0, nz, obody, 0)
    pltpu.sync_copy(u_acc.at[pl.ds(s * VPER + nz * KB, rz)],
                    rows.at[pl.ds(0, rz)])
    pltpu.sync_copy(rows.at[pl.ds(0, rz)],
                    u_out.at[pl.ds(c * NVP + s * VPER + nz * KB, rz)])


def _tc_table_body(x_ref, d_ref, w_ref, b_ref, o_ref):
    d = d_ref[:, 0] + d_ref[:, 1]
    inv = lax.rsqrt(jnp.maximum(d, 1.0))
    h = jnp.dot(x_ref[...], w_ref[...], preferred_element_type=_f32)
    h = jnp.maximum(h + b_ref[...], 0.0)
    o_ref[...] = h * inv[:, None]


def _tc_table(x, d_part, w, b, rows_total, blk):
    grid = rows_total // blk
    f = x.shape[1]
    return pl.pallas_call(
        _tc_table_body,
        grid=(grid,),
        in_specs=[
            pl.BlockSpec((blk, f), lambda i: (i, 0)),
            pl.BlockSpec((blk, 2), lambda i: (i, 0)),
            pl.BlockSpec((f, H), lambda i: (0, 0)),
            pl.BlockSpec((1, H), lambda i: (0, 0)),
        ],
        out_specs=pl.BlockSpec((blk, H), lambda i: (i, 0)),
        out_shape=jax.ShapeDtypeStruct((rows_total, H), _f32),
    )(x, d_part, w, b.reshape(1, H))


def _tc_con_body(sv_ref, ss_ref, dcv_ref, dcs_ref, w_ref, b_ref, o_ref):
    a = lax.rsqrt(jnp.maximum(dcv_ref[:, 0] + dcv_ref[:, 1], 1.0))
    bsc = lax.rsqrt(jnp.maximum(dcs_ref[:, 0] + dcs_ref[:, 1], 1.0))
    t = (a[:, None] * (sv_ref[0] + sv_ref[1])
         + bsc[:, None] * (ss_ref[0] + ss_ref[1]))
    h = jnp.dot(t, w_ref[...], preferred_element_type=_f32)
    h = jnp.maximum(h + 2.0 * b_ref[...], 0.0)
    o_ref[...] = h * a[:, None]


def _tc_con(sv_p, ss_p, dcv_p, dcs_p, w2f, b2f, blk):
    grid = N_CON // blk
    return pl.pallas_call(
        _tc_con_body,
        grid=(grid,),
        in_specs=[
            pl.BlockSpec((2, blk, H), lambda i: (0, i, 0)),
            pl.BlockSpec((2, blk, H), lambda i: (0, i, 0)),
            pl.BlockSpec((blk, 2), lambda i: (i, 0)),
            pl.BlockSpec((blk, 2), lambda i: (i, 0)),
            pl.BlockSpec((H, H), lambda i: (0, 0)),
            pl.BlockSpec((1, H), lambda i: (0, 0)),
        ],
        out_specs=pl.BlockSpec((blk, H), lambda i: (i, 0)),
        out_shape=jax.ShapeDtypeStruct((N_CON, H), _f32),
    )(sv_p, ss_p, dcv_p, dcs_p, w2f, b2f.reshape(1, H))


def _tc_readout_body(u_ref, d_ref, w2b_ref, b2b_ref, w1_ref, b1_ref,
                     w2_ref, b2_ref, w3_ref, b3_ref, o_ref):
    @pl.when(pl.program_id(0) == 0)
    def _():
        o_ref[...] = jnp.zeros_like(o_ref)

    inv = lax.rsqrt(jnp.maximum(d_ref[:, 0] + d_ref[:, 1], 1.0))
    u = (u_ref[0] + u_ref[1]) * inv[:, None]
    h = jnp.maximum(jnp.dot(u, w2b_ref[...], preferred_element_type=_f32)
                    + b2b_ref[...], 0.0)
    l1 = jnp.maximum(jnp.dot(h, w1_ref[...], preferred_element_type=_f32)
                     + b1_ref[...], 0.0)
    l2 = jnp.maximum(jnp.dot(l1, w2_ref[...], preferred_element_type=_f32)
                     + b2_ref[...], 0.0)
    lo = jnp.dot(l2, w3_ref[...], preferred_element_type=_f32) + b3_ref[...]
    o_ref[...] = o_ref[...] + jnp.sum(lo) * (1.0 / N_VAR)


def _tc_readout(u_p, dv_p, w2b, b2b, wo1, bo1, wo2, bo2, wo3, bo3, blk):
    grid = N_VAR // blk
    return pl.pallas_call(
        _tc_readout_body,
        grid=(grid,),
        in_specs=[
            pl.BlockSpec((2, blk, H), lambda i: (0, i, 0)),
            pl.BlockSpec((blk, 2), lambda i: (i, 0)),
            pl.BlockSpec((H, H), lambda i: (0, 0)),
            pl.BlockSpec((1, H), lambda i: (0, 0)),
            pl.BlockSpec((H, H), lambda i: (0, 0)),
            pl.BlockSpec((1, H), lambda i: (0, 0)),
            pl.BlockSpec((H, H), lambda i: (0, 0)),
            pl.BlockSpec((1, H), lambda i: (0, 0)),
            pl.BlockSpec((H, 1), lambda i: (0, 0)),
            pl.BlockSpec((1, 1), lambda i: (0, 0)),
        ],
        out_specs=pl.BlockSpec((1, 1), lambda i: (0, 0)),
        out_shape=jax.ShapeDtypeStruct((1, 1), _f32),
    )(u_p, dv_p, w2b, b2b.reshape(1, H), wo1, bo1.reshape(1, H),
      wo2, bo2.reshape(1, H), wo3, bo3.reshape(1, 1))


def kernel(x_var, x_con, x_soc, v2c_src, v2c_dst, s2c_src, s2c_dst,
           Wv, bv, Wc, bc, Ws, bs,
           W1f, b1f, W1b, b1b, W2f, b2f, W2b, b2b,
           Wo1, bo1, Wo2, bo2, Wo3, bo3):
    del x_con, Wc, bc, W1f, b1f, W1b, b1b
    zeros1 = jnp.zeros((NVP,), _f32)
    zeros2 = jnp.zeros((KV, H), _f32)
    onesv = jnp.ones((KV,), _f32)

    dv_p, ds_p = _sc_deg_src(v2c_src, s2c_src, zeros1, onesv)
    dv_t = dv_p.reshape(NC, NVP).T
    ds_t = ds_p.reshape(NC, NSP).T
    tv = _tc_table(x_var, dv_t, Wv, bv, N_VAR, 2000)
    ts = _tc_table(x_soc, ds_t, Ws, bs, N_SOC, 2000)
    sv_p, dcv_p = _sc_fwd_v(v2c_src, v2c_dst, tv, zeros2, zeros1, onesv)
    ss_p, dcs_p = _sc_fwd_s(s2c_src, s2c_dst, ts, zeros2, zeros1, onesv)
    tc = _tc_con(sv_p.reshape(NC, NCP, H), ss_p.reshape(NC, NCP, H),
                 dcv_p.reshape(NC, NCP).T, dcs_p.reshape(NC, NCP).T,
                 W2f, b2f, 2000)
    u_p = _sc_bwd(v2c_dst, v2c_src, tc, zeros2)
    out = _tc_readout(u_p.reshape(NC, NVP, H), dv_t, W2b, b2b,
                      Wo1, bo1, Wo2, bo2, Wo3, bo3, 2000)
    return out


# speedup vs baseline: 43.3763x; 1.3621x over previous
import functools

import jax
import jax.numpy as jnp
from jax import lax
from jax.experimental import pallas as pl
from jax.experimental.pallas import tpu as pltpu
from jax.experimental.pallas import tpu_sc as plsc

N_VAR, N_CON, N_SOC = 100000, 50000, 10000
E_VC, E_SC = 3200000, 160000
H = 16
NC, NS = 2, 16
NW = NC * NS


def _padded(n):
    per = -(-n // NS)
    per = -(-per // 8) * 8
    return per * NS, per


NVP, VPER = _padded(N_VAR)
NCP, CPER = _padded(N_CON)
NSP, SPER = _padded(N_SOC)

EVW = E_VC // NW
ESW = E_SC // NW
KV = 2000
KB = 800
KS = 1000

_MESH = plsc.VectorSubcoreMesh(core_axis_name="c", subcore_axis_name="s")
_SC_PARAMS = pltpu.CompilerParams(use_tc_tiling_on_sc=False)
_f32 = jnp.float32


@functools.partial(
    pl.kernel,
    out_type=(jax.ShapeDtypeStruct((NC * NVP,), _f32),
              jax.ShapeDtypeStruct((NC * NSP,), _f32)),
    mesh=_MESH,
    compiler_params=_SC_PARAMS,
    scratch_types=[
        pltpu.VMEM_SHARED((NVP,), _f32),
        pltpu.VMEM_SHARED((NSP,), _f32),
        pltpu.VMEM((KV,), jnp.int32),
        pltpu.VMEM((KV,), jnp.int32),
        pltpu.VMEM((KS,), jnp.int32),
        pltpu.VMEM((KV,), _f32),
        pltpu.VMEM((KS,), _f32),
        pltpu.VMEM((VPER,), _f32),
        pltpu.SemaphoreType.DMA,
        pltpu.SemaphoreType.DMA,
    ],
)
def _sc_deg_src(vsrc_hbm, ssrc_hbm, zeros1_hbm, ones_hbm,
                dv_out, ds_out, dv_acc, ds_acc, idxv0, idxv1, idxs,
                onev, ones, stg1, sem_i, sem_s):
    c = lax.axis_index("c")
    s = lax.axis_index("s")
    wid = c * NS + s
    pltpu.sync_copy(zeros1_hbm.at[pl.ds(0, VPER)], stg1)
    pltpu.sync_copy(stg1, dv_acc.at[pl.ds(s * VPER, VPER)])
    pltpu.sync_copy(stg1.at[pl.ds(0, SPER)], ds_acc.at[pl.ds(s * SPER, SPER)])
    pltpu.sync_copy(ones_hbm, onev)
    pltpu.sync_copy(ones_hbm.at[pl.ds(0, KS)], ones)
    plsc.subcore_barrier()

    def _start_idx(i, buf):
        base = pl.multiple_of(wid * EVW + i * KV, 8)
        pltpu.make_async_copy(vsrc_hbm.at[pl.ds(base, KV)], buf, sem_i).start()

    def _wait_idx(buf):
        pltpu.make_async_copy(vsrc_hbm.at[pl.ds(0, KV)], buf, sem_i).wait()

    def _start_scat(buf):
        pltpu.make_async_copy(onev, dv_acc.at[buf], sem_s).start(add=True)

    def _wait_scat(buf):
        pltpu.make_async_copy(onev, dv_acc.at[buf], sem_s).wait()

    _start_idx(0, idxv0)
    njv = (EVW // KV) // 2

    def bodyv(j, carry):
        i0 = 2 * j
        _wait_idx(idxv0)
        _start_scat(idxv0)

        @pl.when(j > 0)
        def _():
            _wait_scat(idxv1)

        _start_idx(i0 + 1, idxv1)
        _wait_idx(idxv1)
        _start_scat(idxv1)
        _wait_scat(idxv0)

        @pl.when(j < njv - 1)
        def _():
            _start_idx(i0 + 2, idxv0)

        return carry

    lax.fori_loop(0, njv, bodyv, 0)
    _wait_scat(idxv1)

    def bodys(i, carry):
        base = pl.multiple_of(wid * ESW + i * KS, 8)
        pltpu.sync_copy(ssrc_hbm.at[pl.ds(base, KS)], idxs)
        pltpu.sync_copy(ones, ds_acc.at[idxs], add=True)
        return carry

    lax.fori_loop(0, ESW // KS, bodys, 0)
    plsc.subcore_barrier()
    pltpu.sync_copy(dv_acc.at[pl.ds(s * VPER, VPER)], stg1)
    pltpu.sync_copy(stg1, dv_out.at[pl.ds(c * NVP + s * VPER, VPER)])
    pltpu.sync_copy(ds_acc.at[pl.ds(s * SPER, SPER)], stg1.at[pl.ds(0, SPER)])
    pltpu.sync_copy(stg1.at[pl.ds(0, SPER)],
                    ds_out.at[pl.ds(c * NSP + s * SPER, SPER)])


@functools.partial(
    pl.kernel,
    out_type=(jax.ShapeDtypeStruct((NC * NCP, H), _f32),
              jax.ShapeDtypeStruct((NC * NCP,), _f32)),
    mesh=_MESH,
    compiler_params=_SC_PARAMS,
    scratch_types=[
        pltpu.VMEM_SHARED((NCP, H), _f32),
        pltpu.VMEM_SHARED((NCP,), _f32),
        pltpu.VMEM((KV,), jnp.int32),
        pltpu.VMEM((KV,), jnp.int32),
        pltpu.VMEM((KV,), jnp.int32),
        pltpu.VMEM((KV,), jnp.int32),
        pltpu.VMEM((KV, H), _f32),
        pltpu.VMEM((KV, H), _f32),
        pltpu.VMEM((KV,), _f32),
        pltpu.VMEM((CPER,), _f32),
        pltpu.SemaphoreType.DMA,
        pltpu.SemaphoreType.DMA,
        pltpu.SemaphoreType.DMA,
    ],
)
def _sc_fwd_v(src_hbm, dst_hbm, tab_hbm, zeros2_hbm, zeros1_hbm, ones_hbm,
              agg_out, deg_out, agg_acc, deg_acc,
              sidx0, didx0, sidx1, didx1, rows0, rows1, onev, stg1,
              sem_i, sem_g, sem_s):
    c = lax.axis_index("c")
    s = lax.axis_index("s")
    wid = c * NS + s
    pltpu.sync_copy(zeros2_hbm.at[pl.ds(0, KV)], rows0)
    pltpu.sync_copy(rows0, agg_acc.at[pl.ds(s * CPER, KV)])
    pltpu.sync_copy(rows0.at[pl.ds(0, CPER - KV)],
                    agg_acc.at[pl.ds(s * CPER + KV, CPER - KV)])
    pltpu.sync_copy(zeros1_hbm.at[pl.ds(0, CPER)], stg1)
    pltpu.sync_copy(stg1, deg_acc.at[pl.ds(s * CPER, CPER)])
    pltpu.sync_copy(ones_hbm, onev)
    plsc.subcore_barrier()

    def _start_idx(i, bs, bd):
        base = pl.multiple_of(wid * EVW + i * KV, 8)
        pltpu.make_async_copy(src_hbm.at[pl.ds(base, KV)], bs, sem_i).start()
        pltpu.make_async_copy(dst_hbm.at[pl.ds(base, KV)], bd, sem_i).start()

    def _wait_idx(bs, bd):
        pltpu.make_async_copy(src_hbm.at[pl.ds(0, KV)], bs, sem_i).wait()
        pltpu.make_async_copy(dst_hbm.at[pl.ds(0, KV)], bd, sem_i).wait()

    def _start_gather(bs, rows):
        pltpu.make_async_copy(tab_hbm.at[bs], rows, sem_g).start()

    def _wait_gather(bs, rows):
        pltpu.make_async_copy(tab_hbm.at[bs], rows, sem_g).wait()

    def _start_scat(rows, bd):
        pltpu.make_async_copy(rows, agg_acc.at[bd], sem_s).start(add=True)
        pltpu.make_async_copy(onev, deg_acc.at[bd], sem_s).start(add=True)

    def _wait_scat(rows, bd):
        pltpu.make_async_copy(rows, agg_acc.at[bd], sem_s).wait()
        pltpu.make_async_copy(onev, deg_acc.at[bd], sem_s).wait()

    nj = (EVW // KV) // 2
    _start_idx(0, sidx0, didx0)

    def body(j, carry):
        i0 = 2 * j
        _wait_idx(sidx0, didx0)
        _start_gather(sidx0, rows0)

        @pl.when(j > 0)
        def _():
            _wait_scat(rows1, didx1)

        _start_idx(i0 + 1, sidx1, didx1)
        _wait_gather(sidx0, rows0)
        _start_scat(rows0, didx0)
        _wait_idx(sidx1, didx1)
        _start_gather(sidx1, rows1)
        _wait_scat(rows0, didx0)

        @pl.when(j < nj - 1)
        def _():
            _start_idx(i0 + 2, sidx0, didx0)

        _wait_gather(sidx1, rows1)
        _start_scat(rows1, didx1)
        return carry

    lax.fori_loop(0, nj, body, 0)
    _wait_scat(rows1, didx1)
    plsc.subcore_barrier()
    pltpu.sync_copy(agg_acc.at[pl.ds(s * CPER, KV)], rows0)
    pltpu.sync_copy(rows0, agg_out.at[pl.ds(c * NCP + s * CPER, KV)])
    pltpu.sync_copy(agg_acc.at[pl.ds(s * CPER + KV, CPER - KV)],
                    rows0.at[pl.ds(0, CPER - KV)])
    pltpu.sync_copy(rows0.at[pl.ds(0, CPER - KV)],
                    agg_out.at[pl.ds(c * NCP + s * CPER + KV, CPER - KV)])
    pltpu.sync_copy(deg_acc.at[pl.ds(s * CPER, CPER)], stg1)
    pltpu.sync_copy(stg1, deg_out.at[pl.ds(c * NCP + s * CPER, CPER)])


@functools.partial(
    pl.kernel,
    out_type=(jax.ShapeDtypeStruct((NC * NCP, H), _f32),
              jax.ShapeDtypeStruct((NC * NCP,), _f32)),
    mesh=_MESH,
    compiler_params=_SC_PARAMS,
    scratch_types=[
        pltpu.VMEM_SHARED((NCP, H), _f32),
        pltpu.VMEM_SHARED((NCP,), _f32),
        pltpu.VMEM((KS,), jnp.int32),
        pltpu.VMEM((KS,), jnp.int32),
        pltpu.VMEM((KS, H), _f32),
        pltpu.VMEM((KS,), _f32),
        pltpu.VMEM((CPER,), _f32),
        pltpu.SemaphoreType.DMA,
    ],
)
def _sc_fwd_s(src_hbm, dst_hbm, tab_hbm, zeros2_hbm, zeros1_hbm, ones_hbm,
              agg_out, deg_out, agg_acc, deg_acc,
              sidx, didx, rows, onev, stg1, sem):
    c = lax.axis_index("c")
    s = lax.axis_index("s")
    wid = c * NS + s
    nz = CPER // KS
    rz = CPER - nz * KS
    pltpu.sync_copy(zeros2_hbm.at[pl.ds(0, KS)], rows)

    def zbody(k, carry):
        off = pl.multiple_of(s * CPER + k * KS, 8)
        pltpu.sync_copy(rows, agg_acc.at[pl.ds(off, KS)])
        return carry

    lax.fori_loop(0, nz, zbody, 0)
    pltpu.sync_copy(rows.at[pl.ds(0, rz)],
                    agg_acc.at[pl.ds(s * CPER + nz * KS, rz)])
    pltpu.sync_copy(zeros1_hbm.at[pl.ds(0, CPER)], stg1)
    pltpu.sync_copy(stg1, deg_acc.at[pl.ds(s * CPER, CPER)])
    pltpu.sync_copy(ones_hbm.at[pl.ds(0, KS)], onev)
    plsc.subcore_barrier()

    def body(i, carry):
        base = pl.multiple_of(wid * ESW + i * KS, 8)
        pltpu.sync_copy(src_hbm.at[pl.ds(base, KS)], sidx)
        pltpu.sync_copy(dst_hbm.at[pl.ds(base, KS)], didx)
        pltpu.async_copy(tab_hbm.at[sidx], rows, sem).wait()
        pltpu.sync_copy(rows, agg_acc.at[didx], add=True)
        pltpu.sync_copy(onev, deg_acc.at[didx], add=True)
        return carry

    lax.fori_loop(0, ESW // KS, body, 0)
    plsc.subcore_barrier()

    def obody(k, carry):
        off = pl.multiple_of(s * CPER + k * KS, 8)
        off2 = pl.multiple_of(c * NCP + s * CPER + k * KS, 8)
        pltpu.sync_copy(agg_acc.at[pl.ds(off, KS)], rows)
        pltpu.sync_copy(rows, agg_out.at[pl.ds(off2, KS)])
        return carry

    lax.fori_loop(0, nz, obody, 0)
    pltpu.sync_copy(agg_acc.at[pl.ds(s * CPER + nz * KS, rz)],
                    rows.at[pl.ds(0, rz)])
    pltpu.sync_copy(rows.at[pl.ds(0, rz)],
                    agg_out.at[pl.ds(c * NCP + s * CPER + nz * KS, rz)])
    pltpu.sync_copy(deg_acc.at[pl.ds(s * CPER, CPER)], stg1)
    pltpu.sync_copy(stg1, deg_out.at[pl.ds(c * NCP + s * CPER, CPER)])


@functools.partial(
    pl.kernel,
    out_type=jax.ShapeDtypeStruct((NC * NVP, H), _f32),
    mesh=_MESH,
    compiler_params=_SC_PARAMS,
    scratch_types=[
        pltpu.VMEM_SHARED((NVP, H), _f32),
        pltpu.VMEM((KB,), jnp.int32),
        pltpu.VMEM((KB,), jnp.int32),
        pltpu.VMEM((KB,), jnp.int32),
        pltpu.VMEM((KB,), jnp.int32),
        pltpu.VMEM((KB, H), _f32),
        pltpu.VMEM((KB, H), _f32),
        pltpu.SemaphoreType.DMA,
        pltpu.SemaphoreType.DMA,
        pltpu.SemaphoreType.DMA,
    ],
)
def _sc_bwd(gidx_hbm, sidx_hbm, tab_hbm, zeros2_hbm,
            u_out, u_acc, gidx0, sidx0, gidx1, sidx1, rows0, rows1,
            sem_i, sem_g, sem_s):
    c = lax.axis_index("c")
    s = lax.axis_index("s")
    wid = c * NS + s
    nz = VPER // KB
    rz = VPER - nz * KB
    pltpu.sync_copy(zeros2_hbm.at[pl.ds(0, KB)], rows0)

    def zbody(k, carry):
        off = pl.multiple_of(s * VPER + k * KB, 8)
        pltpu.sync_copy(rows0, u_acc.at[pl.ds(off, KB)])
        return carry

    lax.fori_loop(0, nz, zbody, 0)
    pltpu.sync_copy(rows0.at[pl.ds(0, rz)],
                    u_acc.at[pl.ds(s * VPER + nz * KB, rz)])
    plsc.subcore_barrier()

    def _start_idx(i, bg, bs):
        base = pl.multiple_of(wid * EVW + i * KB, 8)
        pltpu.make_async_copy(gidx_hbm.at[pl.ds(base, KB)], bg, sem_i).start()
        pltpu.make_async_copy(sidx_hbm.at[pl.ds(base, KB)], bs, sem_i).start()

    def _wait_idx(bg, bs):
        pltpu.make_async_copy(gidx_hbm.at[pl.ds(0, KB)], bg, sem_i).wait()
        pltpu.make_async_copy(sidx_hbm.at[pl.ds(0, KB)], bs, sem_i).wait()

    def _start_gather(bg, rows):
        pltpu.make_async_copy(tab_hbm.at[bg], rows, sem_g).start()

    def _wait_gather(bg, rows):
        pltpu.make_async_copy(tab_hbm.at[bg], rows, sem_g).wait()

    def _start_scat(rows, bs):
        pltpu.make_async_copy(rows, u_acc.at[bs], sem_s).start(add=True)

    def _wait_scat(rows, bs):
        pltpu.make_async_copy(rows, u_acc.at[bs], sem_s).wait()

    ni = EVW // KB
    nj = ni // 2
    _start_idx(0, gidx0, sidx0)

    def body(j, carry):
        i0 = 2 * j
        _wait_idx(gidx0, sidx0)
        _start_gather(gidx0, rows0)

        @pl.when(j > 0)
        def _():
            _wait_scat(rows1, sidx1)

        _start_idx(i0 + 1, gidx1, sidx1)
        _wait_gather(gidx0, rows0)
        _start_scat(rows0, sidx0)
        _wait_idx(gidx1, sidx1)
        _start_gather(gidx1, rows1)
        _wait_scat(rows0, sidx0)
        _start_idx(i0 + 2, gidx0, sidx0)
        _wait_gather(gidx1, rows1)
        _start_scat(rows1, sidx1)
        return carry

    lax.fori_loop(0, nj, body, 0)
    _wait_idx(gidx0, sidx0)
    _start_gather(gidx0, rows0)
    _wait_scat(rows1, sidx1)
    _wait_gather(gidx0, rows0)
    _start_scat(rows0, sidx0)
    _wait_scat(rows0, sidx0)
    plsc.subcore_barrier()

    def obody(k, carry):
        off = pl.multiple_of(s * VPER + k * KB, 8)
        off2 = pl.multiple_of(c * NVP + s * VPER + k * KB, 8)
        pltpu.sync_copy(u_acc.at[pl.ds(off, KB)], rows0)
        pltpu.sync_copy(rows0, u_out.at[pl.ds(off2, KB)])
        return carry

    lax.fori_loop(0, nz, obody, 0)
    pltpu.sync_copy(u_acc.at[pl.ds(s * VPER + nz * KB, rz)],
                    rows0.at[pl.ds(0, rz)])
    pltpu.sync_copy(rows0.at[pl.ds(0, rz)],
                    u_out.at[pl.ds(c * NVP + s * VPER + nz * KB, rz)])


def _tc_table_body(x_ref, d_ref, w_ref, b_ref, o_ref):
    d = d_ref[:, 0] + d_ref[:, 1]
    inv = lax.rsqrt(jnp.maximum(d, 1.0))
    h = jnp.dot(x_ref[...], w_ref[...], preferred_element_type=_f32)
    h = jnp.maximum(h + b_ref[...], 0.0)
    o_ref[...] = h * inv[:, None]


def _tc_table(x, d_part, w, b, rows_total, blk):
    grid = rows_total // blk
    f = x.shape[1]
    return pl.pallas_call(
        _tc_table_body,
        grid=(grid,),
        in_specs=[
            pl.BlockSpec((blk, f), lambda i: (i, 0)),
            pl.BlockSpec((blk, 2), lambda i: (i, 0)),
            pl.BlockSpec((f, H), lambda i: (0, 0)),
            pl.BlockSpec((1, H), lambda i: (0, 0)),
        ],
        out_specs=pl.BlockSpec((blk, H), lambda i: (i, 0)),
        out_shape=jax.ShapeDtypeStruct((rows_total, H), _f32),
    )(x, d_part, w, b.reshape(1, H))


def _tc_con_body(sv_ref, ss_ref, dcv_ref, dcs_ref, w_ref, b_ref, o_ref):
    a = lax.rsqrt(jnp.maximum(dcv_ref[:, 0] + dcv_ref[:, 1], 1.0))
    bsc = lax.rsqrt(jnp.maximum(dcs_ref[:, 0] + dcs_ref[:, 1], 1.0))
    t = (a[:, None] * (sv_ref[0] + sv_ref[1])
         + bsc[:, None] * (ss_ref[0] + ss_ref[1]))
    h = jnp.dot(t, w_ref[...], preferred_element_type=_f32)
    h = jnp.maximum(h + 2.0 * b_ref[...], 0.0)
    o_ref[...] = h * a[:, None]


def _tc_con(sv_p, ss_p, dcv_p, dcs_p, w2f, b2f, blk):
    grid = N_CON // blk
    return pl.pallas_call(
        _tc_con_body,
        grid=(grid,),
        in_specs=[
            pl.BlockSpec((2, blk, H), lambda i: (0, i, 0)),
            pl.BlockSpec((2, blk, H), lambda i: (0, i, 0)),
            pl.BlockSpec((blk, 2), lambda i: (i, 0)),
            pl.BlockSpec((blk, 2), lambda i: (i, 0)),
            pl.BlockSpec((H, H), lambda i: (0, 0)),
            pl.BlockSpec((1, H), lambda i: (0, 0)),
        ],
        out_specs=pl.BlockSpec((blk, H), lambda i: (i, 0)),
        out_shape=jax.ShapeDtypeStruct((N_CON, H), _f32),
    )(sv_p, ss_p, dcv_p, dcs_p, w2f, b2f.reshape(1, H))


def _tc_readout_body(u_ref, d_ref, w2b_ref, b2b_ref, w1_ref, b1_ref,
                     w2_ref, b2_ref, w3_ref, b3_ref, o_ref):
    @pl.when(pl.program_id(0) == 0)
    def _():
        o_ref[...] = jnp.zeros_like(o_ref)

    inv = lax.rsqrt(jnp.maximum(d_ref[:, 0] + d_ref[:, 1], 1.0))
    u = (u_ref[0] + u_ref[1]) * inv[:, None]
    h = jnp.maximum(jnp.dot(u, w2b_ref[...], preferred_element_type=_f32)
                    + b2b_ref[...], 0.0)
    l1 = jnp.maximum(jnp.dot(h, w1_ref[...], preferred_element_type=_f32)
                     + b1_ref[...], 0.0)
    l2 = jnp.maximum(jnp.dot(l1, w2_ref[...], preferred_element_type=_f32)
                     + b2_ref[...], 0.0)
    lo = jnp.dot(l2, w3_ref[...], preferred_element_type=_f32) + b3_ref[...]
    o_ref[...] = o_ref[...] + jnp.sum(lo) * (1.0 / N_VAR)


def _tc_readout(u_p, dv_p, w2b, b2b, wo1, bo1, wo2, bo2, wo3, bo3, blk):
    grid = N_VAR // blk
    return pl.pallas_call(
        _tc_readout_body,
        grid=(grid,),
        in_specs=[
            pl.BlockSpec((2, blk, H), lambda i: (0, i, 0)),
            pl.BlockSpec((blk, 2), lambda i: (i, 0)),
            pl.BlockSpec((H, H), lambda i: (0, 0)),
            pl.BlockSpec((1, H), lambda i: (0, 0)),
            pl.BlockSpec((H, H), lambda i: (0, 0)),
            pl.BlockSpec((1, H), lambda i: (0, 0)),
            pl.BlockSpec((H, H), lambda i: (0, 0)),
            pl.BlockSpec((1, H), lambda i: (0, 0)),
            pl.BlockSpec((H, 1), lambda i: (0, 0)),
            pl.BlockSpec((1, 1), lambda i: (0, 0)),
        ],
        out_specs=pl.BlockSpec((1, 1), lambda i: (0, 0)),
        out_shape=jax.ShapeDtypeStruct((1, 1), _f32),
    )(u_p, dv_p, w2b, b2b.reshape(1, H), wo1, bo1.reshape(1, H),
      wo2, bo2.reshape(1, H), wo3, bo3.reshape(1, 1))


def kernel(x_var, x_con, x_soc, v2c_src, v2c_dst, s2c_src, s2c_dst,
           Wv, bv, Wc, bc, Ws, bs,
           W1f, b1f, W1b, b1b, W2f, b2f, W2b, b2b,
           Wo1, bo1, Wo2, bo2, Wo3, bo3):
    del x_con, Wc, bc, W1f, b1f, W1b, b1b
    zeros1 = jnp.zeros((NVP,), _f32)
    zeros2 = jnp.zeros((KV, H), _f32)
    onesv = jnp.ones((KV,), _f32)

    dv_p, ds_p = _sc_deg_src(v2c_src, s2c_src, zeros1, onesv)
    dv_t = dv_p.reshape(NC, NVP).T
    ds_t = ds_p.reshape(NC, NSP).T
    tv = _tc_table(x_var, dv_t, Wv, bv, N_VAR, 2000)
    ts = _tc_table(x_soc, ds_t, Ws, bs, N_SOC, 2000)
    sv_p, dcv_p = _sc_fwd_v(v2c_src, v2c_dst, tv, zeros2, zeros1, onesv)
    ss_p, dcs_p = _sc_fwd_s(s2c_src, s2c_dst, ts, zeros2, zeros1, onesv)
    tc = _tc_con(sv_p.reshape(NC, NCP, H), ss_p.reshape(NC, NCP, H),
                 dcv_p.reshape(NC, NCP).T, dcs_p.reshape(NC, NCP).T,
                 W2f, b2f, 2000)
    u_p = _sc_bwd(v2c_dst, v2c_src, tc, zeros2)
    out = _tc_readout(u_p.reshape(NC, NVP, H), dv_t, W2b, b2b,
                      Wo1, bo1, Wo2, bo2, Wo3, bo3, 2000)
    return out


# speedup vs baseline: 58.3567x; 1.3454x over previous
import functools

import jax
import jax.numpy as jnp
from jax import lax
from jax.experimental import pallas as pl
from jax.experimental.pallas import tpu as pltpu
from jax.experimental.pallas import tpu_sc as plsc

N_VAR, N_CON, N_SOC = 100000, 50000, 10000
E_VC, E_SC = 3200000, 160000
H = 16
NC, NS = 2, 16
NW = NC * NS


def _padded(n):
    per = -(-n // NS)
    per = -(-per // 8) * 8
    return per * NS, per


NVP, VPER = _padded(N_VAR)
NCP, CPER = _padded(N_CON)
NSP, SPER = _padded(N_SOC)

EVW = E_VC // NW
ESW = E_SC // NW
KV = 2000
KB = 800
KS = 1000

_MESH = plsc.VectorSubcoreMesh(core_axis_name="c", subcore_axis_name="s")
_SC_PARAMS = pltpu.CompilerParams(use_tc_tiling_on_sc=False)
_f32 = jnp.float32


@functools.partial(
    pl.kernel,
    out_type=(jax.ShapeDtypeStruct((NC * NVP,), _f32),
              jax.ShapeDtypeStruct((NC * NSP,), _f32)),
    mesh=_MESH,
    compiler_params=_SC_PARAMS,
    scratch_types=[
        pltpu.VMEM_SHARED((NVP,), _f32),
        pltpu.VMEM_SHARED((NSP,), _f32),
        pltpu.VMEM((KV,), jnp.int32),
        pltpu.VMEM((KV,), jnp.int32),
        pltpu.VMEM((KS,), jnp.int32),
        pltpu.VMEM((KV,), _f32),
        pltpu.VMEM((KS,), _f32),
        pltpu.VMEM((VPER,), _f32),
        pltpu.SemaphoreType.DMA,
        pltpu.SemaphoreType.DMA,
    ],
)
def _sc_deg_src(vsrc_hbm, ssrc_hbm, zeros1_hbm, ones_hbm,
                dv_out, ds_out, dv_acc, ds_acc, idxv0, idxv1, idxs,
                onev, ones, stg1, sem_i, sem_s):
    c = lax.axis_index("c")
    s = lax.axis_index("s")
    wid = c * NS + s
    pltpu.sync_copy(zeros1_hbm.at[pl.ds(0, VPER)], stg1)
    pltpu.sync_copy(stg1, dv_acc.at[pl.ds(s * VPER, VPER)])
    pltpu.sync_copy(stg1.at[pl.ds(0, SPER)], ds_acc.at[pl.ds(s * SPER, SPER)])
    pltpu.sync_copy(ones_hbm, onev)
    pltpu.sync_copy(ones_hbm.at[pl.ds(0, KS)], ones)
    plsc.subcore_barrier()

    def _start_idx(i, buf):
        base = pl.multiple_of(wid * EVW + i * KV, 8)
        pltpu.make_async_copy(vsrc_hbm.at[pl.ds(base, KV)], buf, sem_i).start()

    def _wait_idx(buf):
        pltpu.make_async_copy(vsrc_hbm.at[pl.ds(0, KV)], buf, sem_i).wait()

    def _start_scat(buf):
        pltpu.make_async_copy(onev, dv_acc.at[buf], sem_s).start(add=True)

    def _wait_scat(buf):
        pltpu.make_async_copy(onev, dv_acc.at[buf], sem_s).wait()

    _start_idx(0, idxv0)
    njv = (EVW // KV) // 2

    def bodyv(j, carry):
        i0 = 2 * j
        _wait_idx(idxv0)
        _start_scat(idxv0)

        @pl.when(j > 0)
        def _():
            _wait_scat(idxv1)

        _start_idx(i0 + 1, idxv1)
        _wait_idx(idxv1)
        _start_scat(idxv1)
        _wait_scat(idxv0)

        @pl.when(j < njv - 1)
        def _():
            _start_idx(i0 + 2, idxv0)

        return carry

    lax.fori_loop(0, njv, bodyv, 0)
    _wait_scat(idxv1)

    def bodys(i, carry):
        base = pl.multiple_of(wid * ESW + i * KS, 8)
        pltpu.sync_copy(ssrc_hbm.at[pl.ds(base, KS)], idxs)
        pltpu.sync_copy(ones, ds_acc.at[idxs], add=True)
        return carry

    lax.fori_loop(0, ESW // KS, bodys, 0)
    plsc.subcore_barrier()
    pltpu.sync_copy(dv_acc.at[pl.ds(s * VPER, VPER)], stg1)
    pltpu.sync_copy(stg1, dv_out.at[pl.ds(c * NVP + s * VPER, VPER)])
    pltpu.sync_copy(ds_acc.at[pl.ds(s * SPER, SPER)], stg1.at[pl.ds(0, SPER)])
    pltpu.sync_copy(stg1.at[pl.ds(0, SPER)],
                    ds_out.at[pl.ds(c * NSP + s * SPER, SPER)])


@functools.partial(
    pl.kernel,
    out_type=(jax.ShapeDtypeStruct((NC * NCP, H), _f32),
              jax.ShapeDtypeStruct((NC * NCP,), _f32)),
    mesh=_MESH,
    compiler_params=_SC_PARAMS,
    scratch_types=[
        pltpu.VMEM_SHARED((NCP, H), _f32),
        pltpu.VMEM_SHARED((NCP,), _f32),
        pltpu.VMEM((KV,), jnp.int32),
        pltpu.VMEM((KV,), jnp.int32),
        pltpu.VMEM((KV,), jnp.int32),
        pltpu.VMEM((KV,), jnp.int32),
        pltpu.VMEM((KV, H), _f32),
        pltpu.VMEM((KV, H), _f32),
        pltpu.VMEM((KV,), _f32),
        pltpu.VMEM((CPER,), _f32),
        pltpu.SemaphoreType.DMA,
        pltpu.SemaphoreType.DMA,
        pltpu.SemaphoreType.DMA,
    ],
)
def _sc_fwd_v(src_hbm, dst_hbm, tab_hbm, zeros2_hbm, zeros1_hbm, ones_hbm,
              agg_out, deg_out, agg_acc, deg_acc,
              sidx0, didx0, sidx1, didx1, rows0, rows1, onev, stg1,
              sem_i, sem_g, sem_s):
    c = lax.axis_index("c")
    s = lax.axis_index("s")
    wid = c * NS + s
    pltpu.sync_copy(zeros2_hbm.at[pl.ds(0, KV)], rows0)
    pltpu.sync_copy(rows0, agg_acc.at[pl.ds(s * CPER, KV)])
    pltpu.sync_copy(rows0.at[pl.ds(0, CPER - KV)],
                    agg_acc.at[pl.ds(s * CPER + KV, CPER - KV)])
    pltpu.sync_copy(zeros1_hbm.at[pl.ds(0, CPER)], stg1)
    pltpu.sync_copy(stg1, deg_acc.at[pl.ds(s * CPER, CPER)])
    pltpu.sync_copy(ones_hbm, onev)
    plsc.subcore_barrier()

    def _start_idx(i, bs, bd):
        base = pl.multiple_of(wid * EVW + i * KV, 8)
        pltpu.make_async_copy(src_hbm.at[pl.ds(base, KV)], bs, sem_i).start()
        pltpu.make_async_copy(dst_hbm.at[pl.ds(base, KV)], bd, sem_i).start()

    def _wait_idx(bs, bd):
        pltpu.make_async_copy(src_hbm.at[pl.ds(0, KV)], bs, sem_i).wait()
        pltpu.make_async_copy(dst_hbm.at[pl.ds(0, KV)], bd, sem_i).wait()

    def _start_gather(bs, rows):
        pltpu.make_async_copy(tab_hbm.at[bs], rows, sem_g).start()

    def _wait_gather(bs, rows):
        pltpu.make_async_copy(tab_hbm.at[bs], rows, sem_g).wait()

    def _start_scat(rows, bd):
        pltpu.make_async_copy(rows, agg_acc.at[bd], sem_s).start(add=True)
        pltpu.make_async_copy(onev, deg_acc.at[bd], sem_s).start(add=True)

    def _wait_scat(rows, bd):
        pltpu.make_async_copy(rows, agg_acc.at[bd], sem_s).wait()
        pltpu.make_async_copy(onev, deg_acc.at[bd], sem_s).wait()

    nj = (EVW // KV) // 2
    _start_idx(0, sidx0, didx0)

    def body(j, carry):
        i0 = 2 * j
        _wait_idx(sidx0, didx0)
        _start_gather(sidx0, rows0)

        @pl.when(j > 0)
        def _():
            _wait_scat(rows1, didx1)

        _start_idx(i0 + 1, sidx1, didx1)
        _wait_gather(sidx0, rows0)
        _start_scat(rows0, didx0)
        _wait_idx(sidx1, didx1)
        _start_gather(sidx1, rows1)
        _wait_scat(rows0, didx0)

        @pl.when(j < nj - 1)
        def _():
            _start_idx(i0 + 2, sidx0, didx0)

        _wait_gather(sidx1, rows1)
        _start_scat(rows1, didx1)
        return carry

    lax.fori_loop(0, nj, body, 0)
    _wait_scat(rows1, didx1)
    plsc.subcore_barrier()
    pltpu.sync_copy(agg_acc.at[pl.ds(s * CPER, KV)], rows0)
    pltpu.sync_copy(rows0, agg_out.at[pl.ds(c * NCP + s * CPER, KV)])
    pltpu.sync_copy(agg_acc.at[pl.ds(s * CPER + KV, CPER - KV)],
                    rows0.at[pl.ds(0, CPER - KV)])
    pltpu.sync_copy(rows0.at[pl.ds(0, CPER - KV)],
                    agg_out.at[pl.ds(c * NCP + s * CPER + KV, CPER - KV)])
    pltpu.sync_copy(deg_acc.at[pl.ds(s * CPER, CPER)], stg1)
    pltpu.sync_copy(stg1, deg_out.at[pl.ds(c * NCP + s * CPER, CPER)])


@functools.partial(
    pl.kernel,
    out_type=(jax.ShapeDtypeStruct((NC * NCP, H), _f32),
              jax.ShapeDtypeStruct((NC * NCP,), _f32)),
    mesh=_MESH,
    compiler_params=_SC_PARAMS,
    scratch_types=[
        pltpu.VMEM_SHARED((NCP, H), _f32),
        pltpu.VMEM_SHARED((NCP,), _f32),
        pltpu.VMEM((KS,), jnp.int32),
        pltpu.VMEM((KS,), jnp.int32),
        pltpu.VMEM((KS, H), _f32),
        pltpu.VMEM((KS,), _f32),
        pltpu.VMEM((CPER,), _f32),
        pltpu.SemaphoreType.DMA,
    ],
)
def _sc_fwd_s(src_hbm, dst_hbm, tab_hbm, zeros2_hbm, zeros1_hbm, ones_hbm,
              agg_out, deg_out, agg_acc, deg_acc,
              sidx, didx, rows, onev, stg1, sem):
    c = lax.axis_index("c")
    s = lax.axis_index("s")
    wid = c * NS + s
    nz = CPER // KS
    rz = CPER - nz * KS
    pltpu.sync_copy(zeros2_hbm.at[pl.ds(0, KS)], rows)

    def zbody(k, carry):
        off = pl.multiple_of(s * CPER + k * KS, 8)
        pltpu.sync_copy(rows, agg_acc.at[pl.ds(off, KS)])
        return carry

    lax.fori_loop(0, nz, zbody, 0)
    pltpu.sync_copy(rows.at[pl.ds(0, rz)],
                    agg_acc.at[pl.ds(s * CPER + nz * KS, rz)])
    pltpu.sync_copy(zeros1_hbm.at[pl.ds(0, CPER)], stg1)
    pltpu.sync_copy(stg1, deg_acc.at[pl.ds(s * CPER, CPER)])
    pltpu.sync_copy(ones_hbm.at[pl.ds(0, KS)], onev)
    plsc.subcore_barrier()

    def body(i, carry):
        base = pl.multiple_of(wid * ESW + i * KS, 8)
        pltpu.sync_copy(src_hbm.at[pl.ds(base, KS)], sidx)
        pltpu.sync_copy(dst_hbm.at[pl.ds(base, KS)], didx)
        pltpu.async_copy(tab_hbm.at[sidx], rows, sem).wait()
        pltpu.sync_copy(rows, agg_acc.at[didx], add=True)
        pltpu.sync_copy(onev, deg_acc.at[didx], add=True)
        return carry

    lax.fori_loop(0, ESW // KS, body, 0)
    plsc.subcore_barrier()

    def obody(k, carry):
        off = pl.multiple_of(s * CPER + k * KS, 8)
        off2 = pl.multiple_of(c * NCP + s * CPER + k * KS, 8)
        pltpu.sync_copy(agg_acc.at[pl.ds(off, KS)], rows)
        pltpu.sync_copy(rows, agg_out.at[pl.ds(off2, KS)])
        return carry

    lax.fori_loop(0, nz, obody, 0)
    pltpu.sync_copy(agg_acc.at[pl.ds(s * CPER + nz * KS, rz)],
                    rows.at[pl.ds(0, rz)])
    pltpu.sync_copy(rows.at[pl.ds(0, rz)],
                    agg_out.at[pl.ds(c * NCP + s * CPER + nz * KS, rz)])
    pltpu.sync_copy(deg_acc.at[pl.ds(s * CPER, CPER)], stg1)
    pltpu.sync_copy(stg1, deg_out.at[pl.ds(c * NCP + s * CPER, CPER)])


@functools.partial(
    pl.kernel,
    out_type=jax.ShapeDtypeStruct((NC * NVP, H), _f32),
    mesh=_MESH,
    compiler_params=_SC_PARAMS,
    scratch_types=[
        pltpu.VMEM_SHARED((NVP, H), _f32),
        pltpu.VMEM((KB,), jnp.int32),
        pltpu.VMEM((KB,), jnp.int32),
        pltpu.VMEM((KB,), jnp.int32),
        pltpu.VMEM((KB,), jnp.int32),
        pltpu.VMEM((KB, H), _f32),
        pltpu.VMEM((KB, H), _f32),
        pltpu.SemaphoreType.DMA,
        pltpu.SemaphoreType.DMA,
        pltpu.SemaphoreType.DMA,
    ],
)
def _sc_bwd(gidx_hbm, sidx_hbm, tab_hbm, zeros2_hbm,
            u_out, u_acc, gidx0, sidx0, gidx1, sidx1, rows0, rows1,
            sem_i, sem_g, sem_s):
    c = lax.axis_index("c")
    s = lax.axis_index("s")
    wid = c * NS + s
    nz = VPER // KB
    rz = VPER - nz * KB
    pltpu.sync_copy(zeros2_hbm.at[pl.ds(0, KB)], rows0)

    def zbody(k, carry):
        off = pl.multiple_of(s * VPER + k * KB, 8)
        pltpu.sync_copy(rows0, u_acc.at[pl.ds(off, KB)])
        return carry

    lax.fori_loop(0, nz, zbody, 0)
    pltpu.sync_copy(rows0.at[pl.ds(0, rz)],
                    u_acc.at[pl.ds(s * VPER + nz * KB, rz)])
    plsc.subcore_barrier()

    def _start_idx(i, bg, bs):
        base = pl.multiple_of(wid * EVW + i * KB, 8)
        pltpu.make_async_copy(gidx_hbm.at[pl.ds(base, KB)], bg, sem_i).start()
        pltpu.make_async_copy(sidx_hbm.at[pl.ds(base, KB)], bs, sem_i).start()

    def _wait_idx(bg, bs):
        pltpu.make_async_copy(gidx_hbm.at[pl.ds(0, KB)], bg, sem_i).wait()
        pltpu.make_async_copy(sidx_hbm.at[pl.ds(0, KB)], bs, sem_i).wait()

    def _start_gather(bg, rows):
        pltpu.make_async_copy(tab_hbm.at[bg], rows, sem_g).start()

    def _wait_gather(bg, rows):
        pltpu.make_async_copy(tab_hbm.at[bg], rows, sem_g).wait()

    def _start_scat(rows, bs):
        pltpu.make_async_copy(rows, u_acc.at[bs], sem_s).start(add=True)

    def _wait_scat(rows, bs):
        pltpu.make_async_copy(rows, u_acc.at[bs], sem_s).wait()

    ni = EVW // KB
    nj = ni // 2
    _start_idx(0, gidx0, sidx0)

    def body(j, carry):
        i0 = 2 * j
        _wait_idx(gidx0, sidx0)
        _start_gather(gidx0, rows0)

        @pl.when(j > 0)
        def _():
            _wait_scat(rows1, sidx1)

        _start_idx(i0 + 1, gidx1, sidx1)
        _wait_gather(gidx0, rows0)
        _start_scat(rows0, sidx0)
        _wait_idx(gidx1, sidx1)
        _start_gather(gidx1, rows1)
        _wait_scat(rows0, sidx0)
        _start_idx(i0 + 2, gidx0, sidx0)
        _wait_gather(gidx1, rows1)
        _start_scat(rows1, sidx1)
        return carry

    lax.fori_loop(0, nj, body, 0)
    _wait_idx(gidx0, sidx0)
    _start_gather(gidx0, rows0)
    _wait_scat(rows1, sidx1)
    _wait_gather(gidx0, rows0)
    _start_scat(rows0, sidx0)
    _wait_scat(rows0, sidx0)
    plsc.subcore_barrier()

    def obody(k, carry):
        off = pl.multiple_of(s * VPER + k * KB, 8)
        off2 = pl.multiple_of(c * NVP + s * VPER + k * KB, 8)
        pltpu.sync_copy(u_acc.at[pl.ds(off, KB)], rows0)
        pltpu.sync_copy(rows0, u_out.at[pl.ds(off2, KB)])
        return carry

    lax.fori_loop(0, nz, obody, 0)
    pltpu.sync_copy(u_acc.at[pl.ds(s * VPER + nz * KB, rz)],
                    rows0.at[pl.ds(0, rz)])
    pltpu.sync_copy(rows0.at[pl.ds(0, rz)],
                    u_out.at[pl.ds(c * NVP + s * VPER + nz * KB, rz)])


NV8 = NVP // 8
NC8 = NCP // 8
NS8 = NSP // 8
BLV = 368
GV = NV8 // BLV
GC = NC8 // BLV


def _scale_map(d0, d1, nrow):
    inv = lax.rsqrt(jnp.maximum(d0 + d1, 1.0))
    return jnp.broadcast_to(inv[:, :, None], (nrow, 8, 16)).reshape(nrow, 128)


def _tc_table_body(x_ref, d0_ref, d1_ref, w_ref, b_ref, o_ref):
    m = _scale_map(d0_ref[...], d1_ref[...], o_ref.shape[0])
    h = jnp.dot(x_ref[...], w_ref[...], preferred_element_type=_f32)
    o_ref[...] = jnp.maximum(h + b_ref[...], 0.0) * m


def _tc_table_var(x128, dv_flat, w, b):
    wbd = jnp.kron(jnp.eye(8, dtype=_f32), w)
    bt = jnp.tile(b, 8).reshape(1, 128)
    dv2 = dv_flat.reshape(2 * NV8, 8)
    return pl.pallas_call(
        _tc_table_body,
        grid=(GV,),
        in_specs=[
            pl.BlockSpec((BLV, 72), lambda i: (i, 0)),
            pl.BlockSpec((BLV, 8), lambda i: (i, 0)),
            pl.BlockSpec((BLV, 8), lambda i: (GV + i, 0)),
            pl.BlockSpec((72, 128), lambda i: (0, 0)),
            pl.BlockSpec((1, 128), lambda i: (0, 0)),
        ],
        out_specs=pl.BlockSpec((BLV, 128), lambda i: (i, 0)),
        out_shape=jax.ShapeDtypeStruct((NV8, 128), _f32),
    )(x128, dv2, dv2, wbd, bt)


def _tc_table_soc(x128, ds_flat, w, b):
    wbd = jnp.kron(jnp.eye(8, dtype=_f32), w)
    bt = jnp.tile(b, 8).reshape(1, 128)
    ds2 = ds_flat.reshape(2 * NS8, 8)
    return pl.pallas_call(
        _tc_table_body,
        grid=(1,),
        in_specs=[
            pl.BlockSpec((NS8, 8), lambda i: (0, 0)),
            pl.BlockSpec((NS8, 8), lambda i: (0, 0)),
            pl.BlockSpec((NS8, 8), lambda i: (1, 0)),
            pl.BlockSpec((8, 128), lambda i: (0, 0)),
            pl.BlockSpec((1, 128), lambda i: (0, 0)),
        ],
        out_specs=pl.BlockSpec((NS8, 128), lambda i: (0, 0)),
        out_shape=jax.ShapeDtypeStruct((NS8, 128), _f32),
    )(x128, ds2, ds2, wbd, bt)


def _tc_con_body(sv0_ref, sv1_ref, ss0_ref, ss1_ref,
                 dcv0_ref, dcv1_ref, dcs0_ref, dcs1_ref,
                 w_ref, b_ref, o_ref):
    a = _scale_map(dcv0_ref[...], dcv1_ref[...], o_ref.shape[0])
    bsc = _scale_map(dcs0_ref[...], dcs1_ref[...], o_ref.shape[0])
    t = a * (sv0_ref[...] + sv1_ref[...]) + bsc * (ss0_ref[...] + ss1_ref[...])
    h = jnp.dot(t, w_ref[...], preferred_element_type=_f32)
    o_ref[...] = jnp.maximum(h + 2.0 * b_ref[...], 0.0) * a


def _tc_con(sv_p, ss_p, dcv_p, dcs_p, w2f, b2f):
    wbd = jnp.kron(jnp.eye(8, dtype=_f32), w2f)
    bt = jnp.tile(b2f, 8).reshape(1, 128)
    sv2 = sv_p.reshape(2 * NC8, 128)
    ss2 = ss_p.reshape(2 * NC8, 128)
    dcv2 = dcv_p.reshape(2 * NC8, 8)
    dcs2 = dcs_p.reshape(2 * NC8, 8)
    return pl.pallas_call(
        _tc_con_body,
        grid=(GC,),
        in_specs=[
            pl.BlockSpec((BLV, 128), lambda i: (i, 0)),
            pl.BlockSpec((BLV, 128), lambda i: (GC + i, 0)),
            pl.BlockSpec((BLV, 128), lambda i: (i, 0)),
            pl.BlockSpec((BLV, 128), lambda i: (GC + i, 0)),
            pl.BlockSpec((BLV, 8), lambda i: (i, 0)),
            pl.BlockSpec((BLV, 8), lambda i: (GC + i, 0)),
            pl.BlockSpec((BLV, 8), lambda i: (i, 0)),
            pl.BlockSpec((BLV, 8), lambda i: (GC + i, 0)),
            pl.BlockSpec((128, 128), lambda i: (0, 0)),
            pl.BlockSpec((1, 128), lambda i: (0, 0)),
        ],
        out_specs=pl.BlockSpec((BLV, 128), lambda i: (i, 0)),
        out_shape=jax.ShapeDtypeStruct((NC8, 128), _f32),
    )(sv2, sv2, ss2, ss2, dcv2, dcv2, dcs2, dcs2, wbd, bt)


def _tc_readout_body(u0_ref, u1_ref, d0_ref, d1_ref, w2b_ref, b2b_ref,
                     w1_ref, b1_ref, w2_ref, b2_ref, w3_ref, b3_ref, o_ref):
    i = pl.program_id(0)

    @pl.when(i == 0)
    def _():
        o_ref[...] = jnp.zeros_like(o_ref)

    m = _scale_map(d0_ref[...], d1_ref[...], BLV)
    u = (u0_ref[...] + u1_ref[...]) * m
    h = jnp.maximum(jnp.dot(u, w2b_ref[...], preferred_element_type=_f32)
                    + b2b_ref[...], 0.0)
    l1 = jnp.maximum(jnp.dot(h, w1_ref[...], preferred_element_type=_f32)
                     + b1_ref[...], 0.0)
    l2 = jnp.maximum(jnp.dot(l1, w2_ref[...], preferred_element_type=_f32)
                     + b2_ref[...], 0.0)
    lo = jnp.dot(l2, w3_ref[...], preferred_element_type=_f32) + b3_ref[...]
    r_ids = (8 * (BLV * i + lax.broadcasted_iota(jnp.int32, (BLV, 8), 0))
             + lax.broadcasted_iota(jnp.int32, (BLV, 8), 1))
    lo = jnp.where(r_ids < N_VAR, lo, 0.0)
    o_ref[...] = o_ref[...] + jnp.sum(lo) * (1.0 / N_VAR)


def _tc_readout(u_p, dv_flat, w2b, b2b, wo1, bo1, wo2, bo2, wo3, bo3):
    eye8 = jnp.eye(8, dtype=_f32)
    u2 = u_p.reshape(2 * NV8, 128)
    dv2 = dv_flat.reshape(2 * NV8, 8)
    return pl.pallas_call(
        _tc_readout_body,
        grid=(GV,),
        in_specs=[
            pl.BlockSpec((BLV, 128), lambda i: (i, 0)),
            pl.BlockSpec((BLV, 128), lambda i: (GV + i, 0)),
            pl.BlockSpec((BLV, 8), lambda i: (i, 0)),
            pl.BlockSpec((BLV, 8), lambda i: (GV + i, 0)),
            pl.BlockSpec((128, 128), lambda i: (0, 0)),
            pl.BlockSpec((1, 128), lambda i: (0, 0)),
            pl.BlockSpec((128, 128), lambda i: (0, 0)),
            pl.BlockSpec((1, 128), lambda i: (0, 0)),
            pl.BlockSpec((128, 128), lambda i: (0, 0)),
            pl.BlockSpec((1, 128), lambda i: (0, 0)),
            pl.BlockSpec((128, 8), lambda i: (0, 0)),
            pl.BlockSpec((1, 8), lambda i: (0, 0)),
        ],
        out_specs=pl.BlockSpec((1, 1), lambda i: (0, 0)),
        out_shape=jax.ShapeDtypeStruct((1, 1), _f32),
    )(u2, u2, dv2, dv2,
      jnp.kron(eye8, w2b), jnp.tile(b2b, 8).reshape(1, 128),
      jnp.kron(eye8, wo1), jnp.tile(bo1, 8).reshape(1, 128),
      jnp.kron(eye8, wo2), jnp.tile(bo2, 8).reshape(1, 128),
      jnp.kron(eye8, wo3), jnp.tile(bo3, 8).reshape(1, 8))


def kernel(x_var, x_con, x_soc, v2c_src, v2c_dst, s2c_src, s2c_dst,
           Wv, bv, Wc, bc, Ws, bs,
           W1f, b1f, W1b, b1b, W2f, b2f, W2b, b2b,
           Wo1, bo1, Wo2, bo2, Wo3, bo3):
    del x_con, Wc, bc, W1f, b1f, W1b, b1b
    zeros1 = jnp.zeros((NVP,), _f32)
    zeros2 = jnp.zeros((KV, H), _f32)
    onesv = jnp.ones((KV,), _f32)
    xv128 = jnp.pad(x_var, ((0, NVP - N_VAR), (0, 0))).reshape(NV8, 72)
    xs128 = jnp.pad(x_soc, ((0, NSP - N_SOC), (0, 0))).reshape(NS8, 8)

    dv_p, ds_p = _sc_deg_src(v2c_src, s2c_src, zeros1, onesv)
    tv = _tc_table_var(xv128, dv_p, Wv, bv).reshape(NVP, H)
    ts = _tc_table_soc(xs128, ds_p, Ws, bs).reshape(NSP, H)
    sv_p, dcv_p = _sc_fwd_v(v2c_src, v2c_dst, tv, zeros2, zeros1, onesv)
    ss_p, dcs_p = _sc_fwd_s(s2c_src, s2c_dst, ts, zeros2, zeros1, onesv)
    tc = _tc_con(sv_p, ss_p, dcv_p, dcs_p, W2f, b2f).reshape(NCP, H)
    u_p = _sc_bwd(v2c_dst, v2c_src, tc, zeros2)
    out = _tc_readout(u_p, dv_p, W2b, b2b, Wo1, bo1, Wo2, bo2, Wo3, bo3)
    return out


# speedup vs baseline: 65.7384x; 1.1265x over previous
import functools

import jax
import jax.numpy as jnp
from jax import lax
from jax.experimental import pallas as pl
from jax.experimental.pallas import tpu as pltpu
from jax.experimental.pallas import tpu_sc as plsc

N_VAR, N_CON, N_SOC = 100000, 50000, 10000
E_VC, E_SC = 3200000, 160000
H = 16
NC, NS = 2, 16
NW = NC * NS


def _padded(n):
    per = -(-n // NS)
    per = -(-per // 8) * 8
    return per * NS, per


NVP, VPER = _padded(N_VAR)
NCP, CPER = _padded(N_CON)
NSP, SPER = _padded(N_SOC)

EVW = E_VC // NW
ESW = E_SC // NW
KV = 2000
KB = 800
KS = 1000
KD = 10000

_MESH = plsc.VectorSubcoreMesh(core_axis_name="c", subcore_axis_name="s")
_SC_PARAMS = pltpu.CompilerParams(use_tc_tiling_on_sc=False)
_f32 = jnp.float32


@functools.partial(
    pl.kernel,
    out_type=(jax.ShapeDtypeStruct((NC * NVP,), _f32),
              jax.ShapeDtypeStruct((NC * NSP,), _f32)),
    mesh=_MESH,
    compiler_params=_SC_PARAMS,
    scratch_types=[
        pltpu.VMEM_SHARED((NVP,), _f32),
        pltpu.VMEM_SHARED((NSP,), _f32),
        pltpu.VMEM((KD,), jnp.int32),
        pltpu.VMEM((KD,), jnp.int32),
        pltpu.VMEM((KS,), jnp.int32),
        pltpu.VMEM((KD,), _f32),
        pltpu.VMEM((KS,), _f32),
        pltpu.VMEM((VPER,), _f32),
        pltpu.SemaphoreType.DMA,
        pltpu.SemaphoreType.DMA,
    ],
)
def _sc_deg_src(vsrc_hbm, ssrc_hbm, zeros1_hbm, ones_hbm,
                dv_out, ds_out, dv_acc, ds_acc, idxv0, idxv1, idxs,
                onev, ones, stg1, sem_i, sem_s):
    c = lax.axis_index("c")
    s = lax.axis_index("s")
    wid = c * NS + s
    pltpu.sync_copy(zeros1_hbm.at[pl.ds(0, VPER)], stg1)
    pltpu.sync_copy(stg1, dv_acc.at[pl.ds(s * VPER, VPER)])
    pltpu.sync_copy(stg1.at[pl.ds(0, SPER)], ds_acc.at[pl.ds(s * SPER, SPER)])
    pltpu.sync_copy(ones_hbm, onev)
    pltpu.sync_copy(ones_hbm.at[pl.ds(0, KS)], ones)
    plsc.subcore_barrier()

    def _start_idx(i, buf):
        base = pl.multiple_of(wid * EVW + i * KD, 8)
        pltpu.make_async_copy(vsrc_hbm.at[pl.ds(base, KD)], buf, sem_i).start()

    def _wait_idx(buf):
        pltpu.make_async_copy(vsrc_hbm.at[pl.ds(0, KD)], buf, sem_i).wait()

    def _start_scat(buf):
        pltpu.make_async_copy(onev, dv_acc.at[buf], sem_s).start(add=True)

    def _wait_scat(buf):
        pltpu.make_async_copy(onev, dv_acc.at[buf], sem_s).wait()

    _start_idx(0, idxv0)
    njv = (EVW // KD) // 2

    def bodyv(j, carry):
        i0 = 2 * j
        _wait_idx(idxv0)
        _start_scat(idxv0)

        @pl.when(j > 0)
        def _():
            _wait_scat(idxv1)

        _start_idx(i0 + 1, idxv1)
        _wait_idx(idxv1)
        _start_scat(idxv1)
        _wait_scat(idxv0)

        @pl.when(j < njv - 1)
        def _():
            _start_idx(i0 + 2, idxv0)

        return carry

    lax.fori_loop(0, njv, bodyv, 0)
    _wait_scat(idxv1)

    def bodys(i, carry):
        base = pl.multiple_of(wid * ESW + i * KS, 8)
        pltpu.sync_copy(ssrc_hbm.at[pl.ds(base, KS)], idxs)
        pltpu.sync_copy(ones, ds_acc.at[idxs], add=True)
        return carry

    lax.fori_loop(0, ESW // KS, bodys, 0)
    plsc.subcore_barrier()
    pltpu.sync_copy(dv_acc.at[pl.ds(s * VPER, VPER)], stg1)
    pltpu.sync_copy(stg1, dv_out.at[pl.ds(c * NVP + s * VPER, VPER)])
    pltpu.sync_copy(ds_acc.at[pl.ds(s * SPER, SPER)], stg1.at[pl.ds(0, SPER)])
    pltpu.sync_copy(stg1.at[pl.ds(0, SPER)],
                    ds_out.at[pl.ds(c * NSP + s * SPER, SPER)])


@functools.partial(
    pl.kernel,
    out_type=(jax.ShapeDtypeStruct((NC * NCP, H), _f32),
              jax.ShapeDtypeStruct((NC * NCP,), _f32)),
    mesh=_MESH,
    compiler_params=_SC_PARAMS,
    scratch_types=[
        pltpu.VMEM_SHARED((NCP, H), _f32),
        pltpu.VMEM_SHARED((NCP,), _f32),
        pltpu.VMEM((KV,), jnp.int32),
        pltpu.VMEM((KV,), jnp.int32),
        pltpu.VMEM((KV,), jnp.int32),
        pltpu.VMEM((KV,), jnp.int32),
        pltpu.VMEM((KV, H), _f32),
        pltpu.VMEM((KV, H), _f32),
        pltpu.VMEM((KV,), _f32),
        pltpu.VMEM((CPER,), _f32),
        pltpu.SemaphoreType.DMA,
        pltpu.SemaphoreType.DMA,
        pltpu.SemaphoreType.DMA,
    ],
)
def _sc_fwd_v(src_hbm, dst_hbm, tab_hbm, zeros2_hbm, zeros1_hbm, ones_hbm,
              agg_out, deg_out, agg_acc, deg_acc,
              sidx0, didx0, sidx1, didx1, rows0, rows1, onev, stg1,
              sem_i, sem_g, sem_s):
    c = lax.axis_index("c")
    s = lax.axis_index("s")
    wid = c * NS + s
    pltpu.sync_copy(zeros2_hbm.at[pl.ds(0, KV)], rows0)
    pltpu.sync_copy(rows0, agg_acc.at[pl.ds(s * CPER, KV)])
    pltpu.sync_copy(rows0.at[pl.ds(0, CPER - KV)],
                    agg_acc.at[pl.ds(s * CPER + KV, CPER - KV)])
    pltpu.sync_copy(zeros1_hbm.at[pl.ds(0, CPER)], stg1)
    pltpu.sync_copy(stg1, deg_acc.at[pl.ds(s * CPER, CPER)])
    pltpu.sync_copy(ones_hbm.at[pl.ds(0, KV)], onev)
    plsc.subcore_barrier()

    def _start_idx(i, bs, bd):
        base = pl.multiple_of(wid * EVW + i * KV, 8)
        pltpu.make_async_copy(src_hbm.at[pl.ds(base, KV)], bs, sem_i).start()
        pltpu.make_async_copy(dst_hbm.at[pl.ds(base, KV)], bd, sem_i).start()

    def _wait_idx(bs, bd):
        pltpu.make_async_copy(src_hbm.at[pl.ds(0, KV)], bs, sem_i).wait()
        pltpu.make_async_copy(dst_hbm.at[pl.ds(0, KV)], bd, sem_i).wait()

    def _start_gather(bs, rows):
        pltpu.make_async_copy(tab_hbm.at[bs], rows, sem_g).start()

    def _wait_gather(bs, rows):
        pltpu.make_async_copy(tab_hbm.at[bs], rows, sem_g).wait()

    def _start_scat(rows, bd):
        pltpu.make_async_copy(rows, agg_acc.at[bd], sem_s).start(add=True)
        pltpu.make_async_copy(onev, deg_acc.at[bd], sem_s).start(add=True)

    def _wait_scat(rows, bd):
        pltpu.make_async_copy(rows, agg_acc.at[bd], sem_s).wait()
        pltpu.make_async_copy(onev, deg_acc.at[bd], sem_s).wait()

    nj = (EVW // KV) // 2
    _start_idx(0, sidx0, didx0)

    def body(j, carry):
        i0 = 2 * j
        _wait_idx(sidx0, didx0)
        _start_gather(sidx0, rows0)

        @pl.when(j > 0)
        def _():
            _wait_scat(rows1, didx1)

        _start_idx(i0 + 1, sidx1, didx1)
        _wait_gather(sidx0, rows0)
        _start_scat(rows0, didx0)
        _wait_idx(sidx1, didx1)
        _start_gather(sidx1, rows1)
        _wait_scat(rows0, didx0)

        @pl.when(j < nj - 1)
        def _():
            _start_idx(i0 + 2, sidx0, didx0)

        _wait_gather(sidx1, rows1)
        _start_scat(rows1, didx1)
        return carry

    lax.fori_loop(0, nj, body, 0)
    _wait_scat(rows1, didx1)
    plsc.subcore_barrier()
    pltpu.sync_copy(agg_acc.at[pl.ds(s * CPER, KV)], rows0)
    pltpu.sync_copy(rows0, agg_out.at[pl.ds(c * NCP + s * CPER, KV)])
    pltpu.sync_copy(agg_acc.at[pl.ds(s * CPER + KV, CPER - KV)],
                    rows0.at[pl.ds(0, CPER - KV)])
    pltpu.sync_copy(rows0.at[pl.ds(0, CPER - KV)],
                    agg_out.at[pl.ds(c * NCP + s * CPER + KV, CPER - KV)])
    pltpu.sync_copy(deg_acc.at[pl.ds(s * CPER, CPER)], stg1)
    pltpu.sync_copy(stg1, deg_out.at[pl.ds(c * NCP + s * CPER, CPER)])


@functools.partial(
    pl.kernel,
    out_type=(jax.ShapeDtypeStruct((NC * NCP, H), _f32),
              jax.ShapeDtypeStruct((NC * NCP,), _f32)),
    mesh=_MESH,
    compiler_params=_SC_PARAMS,
    scratch_types=[
        pltpu.VMEM_SHARED((NCP, H), _f32),
        pltpu.VMEM_SHARED((NCP,), _f32),
        pltpu.VMEM((KS,), jnp.int32),
        pltpu.VMEM((KS,), jnp.int32),
        pltpu.VMEM((KS, H), _f32),
        pltpu.VMEM((KS,), _f32),
        pltpu.VMEM((CPER,), _f32),
        pltpu.SemaphoreType.DMA,
    ],
)
def _sc_fwd_s(src_hbm, dst_hbm, tab_hbm, zeros2_hbm, zeros1_hbm, ones_hbm,
              agg_out, deg_out, agg_acc, deg_acc,
              sidx, didx, rows, onev, stg1, sem):
    c = lax.axis_index("c")
    s = lax.axis_index("s")
    wid = c * NS + s
    nz = CPER // KS
    rz = CPER - nz * KS
    pltpu.sync_copy(zeros2_hbm.at[pl.ds(0, KS)], rows)

    def zbody(k, carry):
        off = pl.multiple_of(s * CPER + k * KS, 8)
        pltpu.sync_copy(rows, agg_acc.at[pl.ds(off, KS)])
        return carry

    lax.fori_loop(0, nz, zbody, 0)
    pltpu.sync_copy(rows.at[pl.ds(0, rz)],
                    agg_acc.at[pl.ds(s * CPER + nz * KS, rz)])
    pltpu.sync_copy(zeros1_hbm.at[pl.ds(0, CPER)], stg1)
    pltpu.sync_copy(stg1, deg_acc.at[pl.ds(s * CPER, CPER)])
    pltpu.sync_copy(ones_hbm.at[pl.ds(0, KS)], onev)
    plsc.subcore_barrier()

    def body(i, carry):
        base = pl.multiple_of(wid * ESW + i * KS, 8)
        pltpu.sync_copy(src_hbm.at[pl.ds(base, KS)], sidx)
        pltpu.sync_copy(dst_hbm.at[pl.ds(base, KS)], didx)
        pltpu.async_copy(tab_hbm.at[sidx], rows, sem).wait()
        pltpu.sync_copy(rows, agg_acc.at[didx], add=True)
        pltpu.sync_copy(onev, deg_acc.at[didx], add=True)
        return carry

    lax.fori_loop(0, ESW // KS, body, 0)
    plsc.subcore_barrier()

    def obody(k, carry):
        off = pl.multiple_of(s * CPER + k * KS, 8)
        off2 = pl.multiple_of(c * NCP + s * CPER + k * KS, 8)
        pltpu.sync_copy(agg_acc.at[pl.ds(off, KS)], rows)
        pltpu.sync_copy(rows, agg_out.at[pl.ds(off2, KS)])
        return carry

    lax.fori_loop(0, nz, obody, 0)
    pltpu.sync_copy(agg_acc.at[pl.ds(s * CPER + nz * KS, rz)],
                    rows.at[pl.ds(0, rz)])
    pltpu.sync_copy(rows.at[pl.ds(0, rz)],
                    agg_out.at[pl.ds(c * NCP + s * CPER + nz * KS, rz)])
    pltpu.sync_copy(deg_acc.at[pl.ds(s * CPER, CPER)], stg1)
    pltpu.sync_copy(stg1, deg_out.at[pl.ds(c * NCP + s * CPER, CPER)])


@functools.partial(
    pl.kernel,
    out_type=jax.ShapeDtypeStruct((NC * NVP, H), _f32),
    mesh=_MESH,
    compiler_params=_SC_PARAMS,
    scratch_types=[
        pltpu.VMEM_SHARED((NVP, H), _f32),
        pltpu.VMEM((KB,), jnp.int32),
        pltpu.VMEM((KB,), jnp.int32),
        pltpu.VMEM((KB,), jnp.int32),
        pltpu.VMEM((KB,), jnp.int32),
        pltpu.VMEM((KB, H), _f32),
        pltpu.VMEM((KB, H), _f32),
        pltpu.SemaphoreType.DMA,
        pltpu.SemaphoreType.DMA,
        pltpu.SemaphoreType.DMA,
    ],
)
def _sc_bwd(gidx_hbm, sidx_hbm, tab_hbm, zeros2_hbm,
            u_out, u_acc, gidx0, sidx0, gidx1, sidx1, rows0, rows1,
            sem_i, sem_g, sem_s):
    c = lax.axis_index("c")
    s = lax.axis_index("s")
    wid = c * NS + s
    nz = VPER // KB
    rz = VPER - nz * KB
    pltpu.sync_copy(zeros2_hbm.at[pl.ds(0, KB)], rows0)

    def zbody(k, carry):
        off = pl.multiple_of(s * VPER + k * KB, 8)
        pltpu.sync_copy(rows0, u_acc.at[pl.ds(off, KB)])
        return carry

    lax.fori_loop(0, nz, zbody, 0)
    pltpu.sync_copy(rows0.at[pl.ds(0, rz)],
                    u_acc.at[pl.ds(s * VPER + nz * KB, rz)])
    plsc.subcore_barrier()

    def _start_idx(i, bg, bs):
        base = pl.multiple_of(wid * EVW + i * KB, 8)
        pltpu.make_async_copy(gidx_hbm.at[pl.ds(base, KB)], bg, sem_i).start()
        pltpu.make_async_copy(sidx_hbm.at[pl.ds(base, KB)], bs, sem_i).start()

    def _wait_idx(bg, bs):
        pltpu.make_async_copy(gidx_hbm.at[pl.ds(0, KB)], bg, sem_i).wait()
        pltpu.make_async_copy(sidx_hbm.at[pl.ds(0, KB)], bs, sem_i).wait()

    def _start_gather(bg, rows):
        pltpu.make_async_copy(tab_hbm.at[bg], rows, sem_g).start()

    def _wait_gather(bg, rows):
        pltpu.make_async_copy(tab_hbm.at[bg], rows, sem_g).wait()

    def _start_scat(rows, bs):
        pltpu.make_async_copy(rows, u_acc.at[bs], sem_s).start(add=True)

    def _wait_scat(rows, bs):
        pltpu.make_async_copy(rows, u_acc.at[bs], sem_s).wait()

    ni = EVW // KB
    nj = ni // 2
    _start_idx(0, gidx0, sidx0)

    def body(j, carry):
        i0 = 2 * j
        _wait_idx(gidx0, sidx0)
        _start_gather(gidx0, rows0)

        @pl.when(j > 0)
        def _():
            _wait_scat(rows1, sidx1)

        _start_idx(i0 + 1, gidx1, sidx1)
        _wait_gather(gidx0, rows0)
        _start_scat(rows0, sidx0)
        _wait_idx(gidx1, sidx1)
        _start_gather(gidx1, rows1)
        _wait_scat(rows0, sidx0)
        _start_idx(i0 + 2, gidx0, sidx0)
        _wait_gather(gidx1, rows1)
        _start_scat(rows1, sidx1)
        return carry

    lax.fori_loop(0, nj, body, 0)
    _wait_idx(gidx0, sidx0)
    _start_gather(gidx0, rows0)
    _wait_scat(rows1, sidx1)
    _wait_gather(gidx0, rows0)
    _start_scat(rows0, sidx0)
    _wait_scat(rows0, sidx0)
    plsc.subcore_barrier()

    def obody(k, carry):
        off = pl.multiple_of(s * VPER + k * KB, 8)
        off2 = pl.multiple_of(c * NVP + s * VPER + k * KB, 8)
        pltpu.sync_copy(u_acc.at[pl.ds(off, KB)], rows0)
        pltpu.sync_copy(rows0, u_out.at[pl.ds(off2, KB)])
        return carry

    lax.fori_loop(0, nz, obody, 0)
    pltpu.sync_copy(u_acc.at[pl.ds(s * VPER + nz * KB, rz)],
                    rows0.at[pl.ds(0, rz)])
    pltpu.sync_copy(rows0.at[pl.ds(0, rz)],
                    u_out.at[pl.ds(c * NVP + s * VPER + nz * KB, rz)])


NV8 = NVP // 8
NC8 = NCP // 8
NS8 = NSP // 8


def _emat():
    return jnp.kron(jnp.eye(8, dtype=_f32), jnp.ones((1, 16), _f32))


def _scale_map(d0, d1, e):
    inv = lax.rsqrt(jnp.maximum(d0 + d1, 1.0))
    return jnp.dot(inv, e, preferred_element_type=_f32)


def _tc_embed_body(x_ref, w_ref, b_ref, o_ref):
    h = jnp.dot(x_ref[...], w_ref[...], preferred_element_type=_f32)
    o_ref[...] = jnp.maximum(h + b_ref[...], 0.0)


def _tc_embed_var(x128, w, b):
    wbd = jnp.kron(jnp.eye(8, dtype=_f32), w)
    bt = jnp.tile(b, 8).reshape(1, 128)
    return pl.pallas_call(
        _tc_embed_body,
        grid=(2,),
        in_specs=[
            pl.BlockSpec((NV8 // 2, 72), lambda i: (i, 0)),
            pl.BlockSpec((72, 128), lambda i: (0, 0)),
            pl.BlockSpec((1, 128), lambda i: (0, 0)),
        ],
        out_specs=pl.BlockSpec((NV8 // 2, 128), lambda i: (i, 0)),
        out_shape=jax.ShapeDtypeStruct((NV8, 128), _f32),
    )(x128, wbd, bt)


def _tc_scale_body(t_ref, d0_ref, d1_ref, e_ref, o_ref):
    o_ref[...] = t_ref[...] * _scale_map(d0_ref[...], d1_ref[...], e_ref[...])


def _tc_scale_var(t128, dv_flat):
    dv2 = dv_flat.reshape(2 * NV8, 8)
    return pl.pallas_call(
        _tc_scale_body,
        grid=(2,),
        in_specs=[
            pl.BlockSpec((NV8 // 2, 128), lambda i: (i, 0)),
            pl.BlockSpec((NV8 // 2, 8), lambda i: (i, 0)),
            pl.BlockSpec((NV8 // 2, 8), lambda i: (2 + i, 0)),
            pl.BlockSpec((8, 128), lambda i: (0, 0)),
        ],
        out_specs=pl.BlockSpec((NV8 // 2, 128), lambda i: (i, 0)),
        out_shape=jax.ShapeDtypeStruct((NV8, 128), _f32),
    )(t128, dv2, dv2, _emat())


def _tc_table_soc_body(x_ref, d0_ref, d1_ref, w_ref, b_ref, e_ref, o_ref):
    m = _scale_map(d0_ref[...], d1_ref[...], e_ref[...])
    h = jnp.dot(x_ref[...], w_ref[...], preferred_element_type=_f32)
    o_ref[...] = jnp.maximum(h + b_ref[...], 0.0) * m


def _tc_table_soc(x128, ds_flat, w, b):
    wbd = jnp.kron(jnp.eye(8, dtype=_f32), w)
    bt = jnp.tile(b, 8).reshape(1, 128)
    ds2 = ds_flat.reshape(2 * NS8, 8)
    return pl.pallas_call(
        _tc_table_soc_body,
        grid=(1,),
        in_specs=[
            pl.BlockSpec((NS8, 8), lambda i: (0, 0)),
            pl.BlockSpec((NS8, 8), lambda i: (0, 0)),
            pl.BlockSpec((NS8, 8), lambda i: (1, 0)),
            pl.BlockSpec((8, 128), lambda i: (0, 0)),
            pl.BlockSpec((1, 128), lambda i: (0, 0)),
            pl.BlockSpec((8, 128), lambda i: (0, 0)),
        ],
        out_specs=pl.BlockSpec((NS8, 128), lambda i: (0, 0)),
        out_shape=jax.ShapeDtypeStruct((NS8, 128), _f32),
    )(x128, ds2, ds2, wbd, bt, _emat())


def _tc_con_body(sv0_ref, sv1_ref, ss0_ref, ss1_ref,
                 dcv0_ref, dcv1_ref, dcs0_ref, dcs1_ref,
                 w_ref, b_ref, e_ref, o_ref):
    e = e_ref[...]
    a = _scale_map(dcv0_ref[...], dcv1_ref[...], e)
    bsc = _scale_map(dcs0_ref[...], dcs1_ref[...], e)
    t = a * (sv0_ref[...] + sv1_ref[...]) + bsc * (ss0_ref[...] + ss1_ref[...])
    h = jnp.dot(t, w_ref[...], preferred_element_type=_f32)
    o_ref[...] = jnp.maximum(h + 2.0 * b_ref[...], 0.0) * a


def _tc_con(sv_p, ss_p, dcv_p, dcs_p, w2f, b2f):
    wbd = jnp.kron(jnp.eye(8, dtype=_f32), w2f)
    bt = jnp.tile(b2f, 8).reshape(1, 128)
    sv2 = sv_p.reshape(2 * NC8, 128)
    ss2 = ss_p.reshape(2 * NC8, 128)
    dcv2 = dcv_p.reshape(2 * NC8, 8)
    dcs2 = dcs_p.reshape(2 * NC8, 8)
    return pl.pallas_call(
        _tc_con_body,
        grid=(1,),
        in_specs=[
            pl.BlockSpec((NC8, 128), lambda i: (0, 0)),
            pl.BlockSpec((NC8, 128), lambda i: (1, 0)),
            pl.BlockSpec((NC8, 128), lambda i: (0, 0)),
            pl.BlockSpec((NC8, 128), lambda i: (1, 0)),
            pl.BlockSpec((NC8, 8), lambda i: (0, 0)),
            pl.BlockSpec((NC8, 8), lambda i: (1, 0)),
            pl.BlockSpec((NC8, 8), lambda i: (0, 0)),
            pl.BlockSpec((NC8, 8), lambda i: (1, 0)),
            pl.BlockSpec((128, 128), lambda i: (0, 0)),
            pl.BlockSpec((1, 128), lambda i: (0, 0)),
            pl.BlockSpec((8, 128), lambda i: (0, 0)),
        ],
        out_specs=pl.BlockSpec((NC8, 128), lambda i: (0, 0)),
        out_shape=jax.ShapeDtypeStruct((NC8, 128), _f32),
    )(sv2, sv2, ss2, ss2, dcv2, dcv2, dcs2, dcs2, wbd, bt, _emat())


BLR = NV8 // 2


def _tc_readout_body(u0_ref, u1_ref, d0_ref, d1_ref, w2b_ref, b2b_ref,
                     w1_ref, b1_ref, w2_ref, b2_ref, w3_ref, b3_ref,
                     e_ref, o_ref):
    i = pl.program_id(0)

    @pl.when(i == 0)
    def _():
        o_ref[...] = jnp.zeros_like(o_ref)

    m = _scale_map(d0_ref[...], d1_ref[...], e_ref[...])
    u = (u0_ref[...] + u1_ref[...]) * m
    h = jnp.maximum(jnp.dot(u, w2b_ref[...], preferred_element_type=_f32)
                    + b2b_ref[...], 0.0)
    l1 = jnp.maximum(jnp.dot(h, w1_ref[...], preferred_element_type=_f32)
                     + b1_ref[...], 0.0)
    l2 = jnp.maximum(jnp.dot(l1, w2_ref[...], preferred_element_type=_f32)
                     + b2_ref[...], 0.0)
    lo = jnp.dot(l2, w3_ref[...], preferred_element_type=_f32) + b3_ref[...]
    r_ids = (8 * (BLR * i + lax.broadcasted_iota(jnp.int32, (BLR, 8), 0))
             + lax.broadcasted_iota(jnp.int32, (BLR, 8), 1))
    lo = jnp.where(r_ids < N_VAR, lo, 0.0)
    o_ref[...] = o_ref[...] + jnp.sum(lo) * (1.0 / N_VAR)


def _tc_readout(u_p, dv_flat, w2b, b2b, wo1, bo1, wo2, bo2, wo3, bo3):
    eye8 = jnp.eye(8, dtype=_f32)
    u2 = u_p.reshape(2 * NV8, 128)
    dv2 = dv_flat.reshape(2 * NV8, 8)
    return pl.pallas_call(
        _tc_readout_body,
        grid=(2,),
        in_specs=[
            pl.BlockSpec((BLR, 128), lambda i: (i, 0)),
            pl.BlockSpec((BLR, 128), lambda i: (2 + i, 0)),
            pl.BlockSpec((BLR, 8), lambda i: (i, 0)),
            pl.BlockSpec((BLR, 8), lambda i: (2 + i, 0)),
            pl.BlockSpec((128, 128), lambda i: (0, 0)),
            pl.BlockSpec((1, 128), lambda i: (0, 0)),
            pl.BlockSpec((128, 128), lambda i: (0, 0)),
            pl.BlockSpec((1, 128), lambda i: (0, 0)),
            pl.BlockSpec((128, 128), lambda i: (0, 0)),
            pl.BlockSpec((1, 128), lambda i: (0, 0)),
            pl.BlockSpec((128, 8), lambda i: (0, 0)),
            pl.BlockSpec((1, 8), lambda i: (0, 0)),
            pl.BlockSpec((8, 128), lambda i: (0, 0)),
        ],
        out_specs=pl.BlockSpec((1, 1), lambda i: (0, 0)),
        out_shape=jax.ShapeDtypeStruct((1, 1), _f32),
    )(u2, u2, dv2, dv2,
      jnp.kron(eye8, w2b), jnp.tile(b2b, 8).reshape(1, 128),
      jnp.kron(eye8, wo1), jnp.tile(bo1, 8).reshape(1, 128),
      jnp.kron(eye8, wo2), jnp.tile(bo2, 8).reshape(1, 128),
      jnp.kron(eye8, wo3), jnp.tile(bo3, 8).reshape(1, 8), _emat())


def kernel(x_var, x_con, x_soc, v2c_src, v2c_dst, s2c_src, s2c_dst,
           Wv, bv, Wc, bc, Ws, bs,
           W1f, b1f, W1b, b1b, W2f, b2f, W2b, b2b,
           Wo1, bo1, Wo2, bo2, Wo3, bo3):
    del x_con, Wc, bc, W1f, b1f, W1b, b1b
    zeros1 = jnp.zeros((NVP,), _f32)
    zeros2 = jnp.zeros((KV, H), _f32)
    onesv = jnp.ones((KD,), _f32)
    xv128 = jnp.pad(x_var, ((0, NVP - N_VAR), (0, 0))).reshape(NV8, 72)
    xs128 = jnp.pad(x_soc, ((0, NSP - N_SOC), (0, 0))).reshape(NS8, 8)

    tv0 = _tc_embed_var(xv128, Wv, bv)
    dv_p, ds_p = _sc_deg_src(v2c_src, s2c_src, zeros1, onesv)
    tv = _tc_scale_var(tv0, dv_p).reshape(NVP, H)
    ts = _tc_table_soc(xs128, ds_p, Ws, bs).reshape(NSP, H)
    sv_p, dcv_p = _sc_fwd_v(v2c_src, v2c_dst, tv, zeros2, zeros1, onesv)
    ss_p, dcs_p = _sc_fwd_s(s2c_src, s2c_dst, ts, zeros2, zeros1, onesv)
    tc = _tc_con(sv_p, ss_p, dcv_p, dcs_p, W2f, b2f).reshape(NCP, H)
    u_p = _sc_bwd(v2c_dst, v2c_src, tc, zeros2)
    out = _tc_readout(u_p, dv_p, W2b, b2b, Wo1, bo1, Wo2, bo2, Wo3, bo3)
    return out


# speedup vs baseline: 68.8823x; 1.0478x over previous
import functools

import jax
import jax.numpy as jnp
from jax import lax
from jax.experimental import pallas as pl
from jax.experimental.pallas import tpu as pltpu
from jax.experimental.pallas import tpu_sc as plsc

N_VAR, N_CON, N_SOC = 100000, 50000, 10000
E_VC, E_SC = 3200000, 160000
H = 16
NC, NS = 2, 16
NW = NC * NS


def _padded(n):
    per = -(-n // NS)
    per = -(-per // 8) * 8
    return per * NS, per


NVP, VPER = _padded(N_VAR)
NCP, CPER = _padded(N_CON)
NSP, SPER = _padded(N_SOC)

EVW = E_VC // NW
ESW = E_SC // NW
KV = 2000
KB = 800
KS = 2500
KD = 10000

_MESH = plsc.VectorSubcoreMesh(core_axis_name="c", subcore_axis_name="s")
_SC_PARAMS = pltpu.CompilerParams(use_tc_tiling_on_sc=False)
_f32 = jnp.float32


@functools.partial(
    pl.kernel,
    out_type=(jax.ShapeDtypeStruct((NC * NVP,), _f32),
              jax.ShapeDtypeStruct((NC * NSP,), _f32)),
    mesh=_MESH,
    compiler_params=_SC_PARAMS,
    scratch_types=[
        pltpu.VMEM_SHARED((NVP,), _f32),
        pltpu.VMEM_SHARED((NSP,), _f32),
        pltpu.VMEM((KD,), jnp.int32),
        pltpu.VMEM((KD,), jnp.int32),
        pltpu.VMEM((KS,), jnp.int32),
        pltpu.VMEM((KD,), _f32),
        pltpu.VMEM((KS,), _f32),
        pltpu.VMEM((VPER,), _f32),
        pltpu.SemaphoreType.DMA,
        pltpu.SemaphoreType.DMA,
    ],
)
def _sc_deg_src(vsrc_hbm, ssrc_hbm, zeros1_hbm, ones_hbm,
                dv_out, ds_out, dv_acc, ds_acc, idxv0, idxv1, idxs,
                onev, ones, stg1, sem_i, sem_s):
    c = lax.axis_index("c")
    s = lax.axis_index("s")
    wid = c * NS + s
    pltpu.sync_copy(zeros1_hbm.at[pl.ds(0, VPER)], stg1)
    pltpu.sync_copy(stg1, dv_acc.at[pl.ds(s * VPER, VPER)])
    pltpu.sync_copy(stg1.at[pl.ds(0, SPER)], ds_acc.at[pl.ds(s * SPER, SPER)])
    pltpu.sync_copy(ones_hbm, onev)
    pltpu.sync_copy(ones_hbm.at[pl.ds(0, KS)], ones)
    plsc.subcore_barrier()

    def _start_idx(i, buf):
        base = pl.multiple_of(wid * EVW + i * KD, 8)
        pltpu.make_async_copy(vsrc_hbm.at[pl.ds(base, KD)], buf, sem_i).start()

    def _wait_idx(buf):
        pltpu.make_async_copy(vsrc_hbm.at[pl.ds(0, KD)], buf, sem_i).wait()

    def _start_scat(buf):
        pltpu.make_async_copy(onev, dv_acc.at[buf], sem_s).start(add=True)

    def _wait_scat(buf):
        pltpu.make_async_copy(onev, dv_acc.at[buf], sem_s).wait()

    _start_idx(0, idxv0)
    njv = (EVW // KD) // 2

    def bodyv(j, carry):
        i0 = 2 * j
        _wait_idx(idxv0)
        _start_scat(idxv0)

        @pl.when(j > 0)
        def _():
            _wait_scat(idxv1)

        _start_idx(i0 + 1, idxv1)
        _wait_idx(idxv1)
        _start_scat(idxv1)
        _wait_scat(idxv0)

        @pl.when(j < njv - 1)
        def _():
            _start_idx(i0 + 2, idxv0)

        return carry

    lax.fori_loop(0, njv, bodyv, 0)
    _wait_scat(idxv1)

    def bodys(i, carry):
        base = pl.multiple_of(wid * ESW + i * KS, 8)
        pltpu.sync_copy(ssrc_hbm.at[pl.ds(base, KS)], idxs)
        pltpu.sync_copy(ones, ds_acc.at[idxs], add=True)
        return carry

    lax.fori_loop(0, ESW // KS, bodys, 0)
    plsc.subcore_barrier()
    pltpu.sync_copy(dv_acc.at[pl.ds(s * VPER, VPER)], stg1)
    pltpu.sync_copy(stg1, dv_out.at[pl.ds(c * NVP + s * VPER, VPER)])
    pltpu.sync_copy(ds_acc.at[pl.ds(s * SPER, SPER)], stg1.at[pl.ds(0, SPER)])
    pltpu.sync_copy(stg1.at[pl.ds(0, SPER)],
                    ds_out.at[pl.ds(c * NSP + s * SPER, SPER)])


@functools.partial(
    pl.kernel,
    out_type=(jax.ShapeDtypeStruct((NC * NCP, H), _f32),
              jax.ShapeDtypeStruct((NC * NCP,), _f32)),
    mesh=_MESH,
    compiler_params=_SC_PARAMS,
    scratch_types=[
        pltpu.VMEM_SHARED((NCP, H), _f32),
        pltpu.VMEM_SHARED((NCP,), _f32),
        pltpu.VMEM((KV,), jnp.int32),
        pltpu.VMEM((KV,), jnp.int32),
        pltpu.VMEM((KV,), jnp.int32),
        pltpu.VMEM((KV,), jnp.int32),
        pltpu.VMEM((KV, H), _f32),
        pltpu.VMEM((KV, H), _f32),
        pltpu.VMEM((KV,), _f32),
        pltpu.VMEM((CPER,), _f32),
        pltpu.SemaphoreType.DMA,
        pltpu.SemaphoreType.DMA,
        pltpu.SemaphoreType.DMA,
    ],
)
def _sc_fwd_v(src_hbm, dst_hbm, tab_hbm, zeros2_hbm, zeros1_hbm, ones_hbm,
              agg_out, deg_out, agg_acc, deg_acc,
              sidx0, didx0, sidx1, didx1, rows0, rows1, onev, stg1,
              sem_i, sem_g, sem_s):
    c = lax.axis_index("c")
    s = lax.axis_index("s")
    wid = c * NS + s
    pltpu.sync_copy(zeros2_hbm.at[pl.ds(0, KV)], rows0)
    pltpu.sync_copy(rows0, agg_acc.at[pl.ds(s * CPER, KV)])
    pltpu.sync_copy(rows0.at[pl.ds(0, CPER - KV)],
                    agg_acc.at[pl.ds(s * CPER + KV, CPER - KV)])
    pltpu.sync_copy(zeros1_hbm.at[pl.ds(0, CPER)], stg1)
    pltpu.sync_copy(stg1, deg_acc.at[pl.ds(s * CPER, CPER)])
    pltpu.sync_copy(ones_hbm.at[pl.ds(0, KV)], onev)
    plsc.subcore_barrier()

    def _start_idx(i, bs, bd):
        base = pl.multiple_of(wid * EVW + i * KV, 8)
        pltpu.make_async_copy(src_hbm.at[pl.ds(base, KV)], bs, sem_i).start()
        pltpu.make_async_copy(dst_hbm.at[pl.ds(base, KV)], bd, sem_i).start()

    def _wait_idx(bs, bd):
        pltpu.make_async_copy(src_hbm.at[pl.ds(0, KV)], bs, sem_i).wait()
        pltpu.make_async_copy(dst_hbm.at[pl.ds(0, KV)], bd, sem_i).wait()

    def _start_gather(bs, rows):
        pltpu.make_async_copy(tab_hbm.at[bs], rows, sem_g).start()

    def _wait_gather(bs, rows):
        pltpu.make_async_copy(tab_hbm.at[bs], rows, sem_g).wait()

    def _start_scat(rows, bd):
        pltpu.make_async_copy(rows, agg_acc.at[bd], sem_s).start(add=True)
        pltpu.make_async_copy(onev, deg_acc.at[bd], sem_s).start(add=True)

    def _wait_scat(rows, bd):
        pltpu.make_async_copy(rows, agg_acc.at[bd], sem_s).wait()
        pltpu.make_async_copy(onev, deg_acc.at[bd], sem_s).wait()

    nj = (EVW // KV) // 2
    _start_idx(0, sidx0, didx0)

    def body(j, carry):
        i0 = 2 * j
        _wait_idx(sidx0, didx0)
        _start_gather(sidx0, rows0)

        @pl.when(j > 0)
        def _():
            _wait_scat(rows1, didx1)

        _start_idx(i0 + 1, sidx1, didx1)
        _wait_gather(sidx0, rows0)
        _start_scat(rows0, didx0)
        _wait_idx(sidx1, didx1)
        _start_gather(sidx1, rows1)
        _wait_scat(rows0, didx0)

        @pl.when(j < nj - 1)
        def _():
            _start_idx(i0 + 2, sidx0, didx0)

        _wait_gather(sidx1, rows1)
        _start_scat(rows1, didx1)
        return carry

    lax.fori_loop(0, nj, body, 0)
    _wait_scat(rows1, didx1)
    plsc.subcore_barrier()
    pltpu.sync_copy(agg_acc.at[pl.ds(s * CPER, KV)], rows0)
    pltpu.sync_copy(rows0, agg_out.at[pl.ds(c * NCP + s * CPER, KV)])
    pltpu.sync_copy(agg_acc.at[pl.ds(s * CPER + KV, CPER - KV)],
                    rows0.at[pl.ds(0, CPER - KV)])
    pltpu.sync_copy(rows0.at[pl.ds(0, CPER - KV)],
                    agg_out.at[pl.ds(c * NCP + s * CPER + KV, CPER - KV)])
    pltpu.sync_copy(deg_acc.at[pl.ds(s * CPER, CPER)], stg1)
    pltpu.sync_copy(stg1, deg_out.at[pl.ds(c * NCP + s * CPER, CPER)])


@functools.partial(
    pl.kernel,
    out_type=(jax.ShapeDtypeStruct((NC * NCP, H), _f32),
              jax.ShapeDtypeStruct((NC * NCP,), _f32)),
    mesh=_MESH,
    compiler_params=_SC_PARAMS,
    scratch_types=[
        pltpu.VMEM_SHARED((NCP, H), _f32),
        pltpu.VMEM_SHARED((NCP,), _f32),
        pltpu.VMEM((KS,), jnp.int32),
        pltpu.VMEM((KS,), jnp.int32),
        pltpu.VMEM((KS, H), _f32),
        pltpu.VMEM((KS,), _f32),
        pltpu.VMEM((CPER,), _f32),
        pltpu.SemaphoreType.DMA,
    ],
)
def _sc_fwd_s(src_hbm, dst_hbm, tab_hbm, zeros2_hbm, zeros1_hbm, ones_hbm,
              agg_out, deg_out, agg_acc, deg_acc,
              sidx, didx, rows, onev, stg1, sem):
    c = lax.axis_index("c")
    s = lax.axis_index("s")
    wid = c * NS + s
    nz = CPER // KS
    rz = CPER - nz * KS
    pltpu.sync_copy(zeros2_hbm.at[pl.ds(0, KS)], rows)

    def zbody(k, carry):
        off = pl.multiple_of(s * CPER + k * KS, 8)
        pltpu.sync_copy(rows, agg_acc.at[pl.ds(off, KS)])
        return carry

    lax.fori_loop(0, nz, zbody, 0)
    pltpu.sync_copy(rows.at[pl.ds(0, rz)],
                    agg_acc.at[pl.ds(s * CPER + nz * KS, rz)])
    pltpu.sync_copy(zeros1_hbm.at[pl.ds(0, CPER)], stg1)
    pltpu.sync_copy(stg1, deg_acc.at[pl.ds(s * CPER, CPER)])
    pltpu.sync_copy(ones_hbm.at[pl.ds(0, KS)], onev)
    plsc.subcore_barrier()

    def body(i, carry):
        base = pl.multiple_of(wid * ESW + i * KS, 8)
        pltpu.sync_copy(src_hbm.at[pl.ds(base, KS)], sidx)
        pltpu.sync_copy(dst_hbm.at[pl.ds(base, KS)], didx)
        pltpu.async_copy(tab_hbm.at[sidx], rows, sem).wait()
        pltpu.sync_copy(rows, agg_acc.at[didx], add=True)
        pltpu.sync_copy(onev, deg_acc.at[didx], add=True)
        return carry

    lax.fori_loop(0, ESW // KS, body, 0)
    plsc.subcore_barrier()

    def obody(k, carry):
        off = pl.multiple_of(s * CPER + k * KS, 8)
        off2 = pl.multiple_of(c * NCP + s * CPER + k * KS, 8)
        pltpu.sync_copy(agg_acc.at[pl.ds(off, KS)], rows)
        pltpu.sync_copy(rows, agg_out.at[pl.ds(off2, KS)])
        return carry

    lax.fori_loop(0, nz, obody, 0)
    pltpu.sync_copy(agg_acc.at[pl.ds(s * CPER + nz * KS, rz)],
                    rows.at[pl.ds(0, rz)])
    pltpu.sync_copy(rows.at[pl.ds(0, rz)],
                    agg_out.at[pl.ds(c * NCP + s * CPER + nz * KS, rz)])
    pltpu.sync_copy(deg_acc.at[pl.ds(s * CPER, CPER)], stg1)
    pltpu.sync_copy(stg1, deg_out.at[pl.ds(c * NCP + s * CPER, CPER)])


@functools.partial(
    pl.kernel,
    out_type=jax.ShapeDtypeStruct((NC * NVP, H), _f32),
    mesh=_MESH,
    compiler_params=_SC_PARAMS,
    scratch_types=[
        pltpu.VMEM_SHARED((NVP, H), _f32),
        pltpu.VMEM((KB,), jnp.int32),
        pltpu.VMEM((KB,), jnp.int32),
        pltpu.VMEM((KB,), jnp.int32),
        pltpu.VMEM((KB,), jnp.int32),
        pltpu.VMEM((KB, H), _f32),
        pltpu.VMEM((KB, H), _f32),
        pltpu.SemaphoreType.DMA,
        pltpu.SemaphoreType.DMA,
        pltpu.SemaphoreType.DMA,
    ],
)
def _sc_bwd(gidx_hbm, sidx_hbm, tab_hbm, zeros2_hbm,
            u_out, u_acc, gidx0, sidx0, gidx1, sidx1, rows0, rows1,
            sem_i, sem_g, sem_s):
    c = lax.axis_index("c")
    s = lax.axis_index("s")
    wid = c * NS + s
    nz = VPER // KB
    rz = VPER - nz * KB
    pltpu.sync_copy(zeros2_hbm.at[pl.ds(0, KB)], rows0)

    def zbody(k, carry):
        off = pl.multiple_of(s * VPER + k * KB, 8)
        pltpu.sync_copy(rows0, u_acc.at[pl.ds(off, KB)])
        return carry

    lax.fori_loop(0, nz, zbody, 0)
    pltpu.sync_copy(rows0.at[pl.ds(0, rz)],
                    u_acc.at[pl.ds(s * VPER + nz * KB, rz)])
    plsc.subcore_barrier()

    def _start_idx(i, bg, bs):
        base = pl.multiple_of(wid * EVW + i * KB, 8)
        pltpu.make_async_copy(gidx_hbm.at[pl.ds(base, KB)], bg, sem_i).start()
        pltpu.make_async_copy(sidx_hbm.at[pl.ds(base, KB)], bs, sem_i).start()

    def _wait_idx(bg, bs):
        pltpu.make_async_copy(gidx_hbm.at[pl.ds(0, KB)], bg, sem_i).wait()
        pltpu.make_async_copy(sidx_hbm.at[pl.ds(0, KB)], bs, sem_i).wait()

    def _start_gather(bg, rows):
        pltpu.make_async_copy(tab_hbm.at[bg], rows, sem_g).start()

    def _wait_gather(bg, rows):
        pltpu.make_async_copy(tab_hbm.at[bg], rows, sem_g).wait()

    def _start_scat(rows, bs):
        pltpu.make_async_copy(rows, u_acc.at[bs], sem_s).start(add=True)

    def _wait_scat(rows, bs):
        pltpu.make_async_copy(rows, u_acc.at[bs], sem_s).wait()

    ni = EVW // KB
    nj = ni // 2
    _start_idx(0, gidx0, sidx0)

    def body(j, carry):
        i0 = 2 * j
        _wait_idx(gidx0, sidx0)
        _start_gather(gidx0, rows0)

        @pl.when(j > 0)
        def _():
            _wait_scat(rows1, sidx1)

        _start_idx(i0 + 1, gidx1, sidx1)
        _wait_gather(gidx0, rows0)
        _start_scat(rows0, sidx0)
        _wait_idx(gidx1, sidx1)
        _start_gather(gidx1, rows1)
        _wait_scat(rows0, sidx0)
        _start_idx(i0 + 2, gidx0, sidx0)
        _wait_gather(gidx1, rows1)
        _start_scat(rows1, sidx1)
        return carry

    lax.fori_loop(0, nj, body, 0)
    _wait_idx(gidx0, sidx0)
    _start_gather(gidx0, rows0)
    _wait_scat(rows1, sidx1)
    _wait_gather(gidx0, rows0)
    _start_scat(rows0, sidx0)
    _wait_scat(rows0, sidx0)
    plsc.subcore_barrier()

    def obody(k, carry):
        off = pl.multiple_of(s * VPER + k * KB, 8)
        off2 = pl.multiple_of(c * NVP + s * VPER + k * KB, 8)
        pltpu.sync_copy(u_acc.at[pl.ds(off, KB)], rows0)
        pltpu.sync_copy(rows0, u_out.at[pl.ds(off2, KB)])
        return carry

    lax.fori_loop(0, nz, obody, 0)
    pltpu.sync_copy(u_acc.at[pl.ds(s * VPER + nz * KB, rz)],
                    rows0.at[pl.ds(0, rz)])
    pltpu.sync_copy(rows0.at[pl.ds(0, rz)],
                    u_out.at[pl.ds(c * NVP + s * VPER + nz * KB, rz)])


NV8 = NVP // 8
NC8 = NCP // 8
NVR8 = N_VAR // 8
NSR8 = N_SOC // 8


def _emat():
    return jnp.kron(jnp.eye(8, dtype=_f32), jnp.ones((1, 16), _f32))


def _scale_map(d0, d1, e):
    inv = lax.rsqrt(jnp.maximum(d0 + d1, 1.0))
    return jnp.dot(inv, e, preferred_element_type=_f32)


def _tc_embed_body(x_ref, w_ref, b_ref, o_ref):
    h = jnp.dot(x_ref[...], w_ref[...], preferred_element_type=_f32)
    o_ref[...] = jnp.maximum(h + b_ref[...], 0.0)


def _tc_embed_var(x128, w, b):
    wbd = jnp.kron(jnp.eye(8, dtype=_f32), w)
    bt = jnp.tile(b, 8).reshape(1, 128)
    return pl.pallas_call(
        _tc_embed_body,
        grid=(1,),
        in_specs=[
            pl.BlockSpec((NVR8, 72), lambda i: (0, 0)),
            pl.BlockSpec((72, 128), lambda i: (0, 0)),
            pl.BlockSpec((1, 128), lambda i: (0, 0)),
        ],
        out_specs=pl.BlockSpec((NVR8, 128), lambda i: (0, 0)),
        out_shape=jax.ShapeDtypeStruct((NVR8, 128), _f32),
    )(x128, wbd, bt)


def _tc_scale_body(t_ref, d0_ref, d1_ref, e_ref, o_ref):
    o_ref[...] = t_ref[...] * _scale_map(d0_ref[...], d1_ref[...], e_ref[...])


def _tc_scale_var(t128, dv_flat):
    dva = dv_flat[:N_VAR].reshape(NVR8, 8)
    dvb = dv_flat[NVP:NVP + N_VAR].reshape(NVR8, 8)
    return pl.pallas_call(
        _tc_scale_body,
        grid=(1,),
        in_specs=[
            pl.BlockSpec((NVR8, 128), lambda i: (0, 0)),
            pl.BlockSpec((NVR8, 8), lambda i: (0, 0)),
            pl.BlockSpec((NVR8, 8), lambda i: (0, 0)),
            pl.BlockSpec((8, 128), lambda i: (0, 0)),
        ],
        out_specs=pl.BlockSpec((NVR8, 128), lambda i: (0, 0)),
        out_shape=jax.ShapeDtypeStruct((NVR8, 128), _f32),
    )(t128, dva, dvb, _emat())


def _tc_table_soc_body(x_ref, d0_ref, d1_ref, w_ref, b_ref, e_ref, o_ref):
    m = _scale_map(d0_ref[...], d1_ref[...], e_ref[...])
    h = jnp.dot(x_ref[...], w_ref[...], preferred_element_type=_f32)
    o_ref[...] = jnp.maximum(h + b_ref[...], 0.0) * m


def _tc_table_soc(x128, ds_flat, w, b):
    wbd = jnp.kron(jnp.eye(8, dtype=_f32), w)
    bt = jnp.tile(b, 8).reshape(1, 128)
    dsa = ds_flat[:N_SOC].reshape(NSR8, 8)
    dsb = ds_flat[NSP:NSP + N_SOC].reshape(NSR8, 8)
    return pl.pallas_call(
        _tc_table_soc_body,
        grid=(1,),
        in_specs=[
            pl.BlockSpec((NSR8, 8), lambda i: (0, 0)),
            pl.BlockSpec((NSR8, 8), lambda i: (0, 0)),
            pl.BlockSpec((NSR8, 8), lambda i: (0, 0)),
            pl.BlockSpec((8, 128), lambda i: (0, 0)),
            pl.BlockSpec((1, 128), lambda i: (0, 0)),
            pl.BlockSpec((8, 128), lambda i: (0, 0)),
        ],
        out_specs=pl.BlockSpec((NSR8, 128), lambda i: (0, 0)),
        out_shape=jax.ShapeDtypeStruct((NSR8, 128), _f32),
    )(x128, dsa, dsb, wbd, bt, _emat())


def _tc_con_body(sv0_ref, sv1_ref, ss0_ref, ss1_ref,
                 dcv0_ref, dcv1_ref, dcs0_ref, dcs1_ref,
                 w_ref, b_ref, e_ref, o_ref):
    e = e_ref[...]
    a = _scale_map(dcv0_ref[...], dcv1_ref[...], e)
    bsc = _scale_map(dcs0_ref[...], dcs1_ref[...], e)
    t = a * (sv0_ref[...] + sv1_ref[...]) + bsc * (ss0_ref[...] + ss1_ref[...])
    h = jnp.dot(t, w_ref[...], preferred_element_type=_f32)
    o_ref[...] = jnp.maximum(h + 2.0 * b_ref[...], 0.0) * a


def _tc_con(sv_p, ss_p, dcv_p, dcs_p, w2f, b2f):
    wbd = jnp.kron(jnp.eye(8, dtype=_f32), w2f)
    bt = jnp.tile(b2f, 8).reshape(1, 128)
    sv2 = sv_p.reshape(2 * NC8, 128)
    ss2 = ss_p.reshape(2 * NC8, 128)
    dcv2 = dcv_p.reshape(2 * NC8, 8)
    dcs2 = dcs_p.reshape(2 * NC8, 8)
    return pl.pallas_call(
        _tc_con_body,
        grid=(1,),
        in_specs=[
            pl.BlockSpec((NC8, 128), lambda i: (0, 0)),
            pl.BlockSpec((NC8, 128), lambda i: (1, 0)),
            pl.BlockSpec((NC8, 128), lambda i: (0, 0)),
            pl.BlockSpec((NC8, 128), lambda i: (1, 0)),
            pl.BlockSpec((NC8, 8), lambda i: (0, 0)),
            pl.BlockSpec((NC8, 8), lambda i: (1, 0)),
            pl.BlockSpec((NC8, 8), lambda i: (0, 0)),
            pl.BlockSpec((NC8, 8), lambda i: (1, 0)),
            pl.BlockSpec((128, 128), lambda i: (0, 0)),
            pl.BlockSpec((1, 128), lambda i: (0, 0)),
            pl.BlockSpec((8, 128), lambda i: (0, 0)),
        ],
        out_specs=pl.BlockSpec((NC8, 128), lambda i: (0, 0)),
        out_shape=jax.ShapeDtypeStruct((NC8, 128), _f32),
    )(sv2, sv2, ss2, ss2, dcv2, dcv2, dcs2, dcs2, wbd, bt, _emat())


BLR = NV8 // 2


def _tc_readout_body(u0_ref, u1_ref, d0_ref, d1_ref, w2b_ref, b2b_ref,
                     w1_ref, b1_ref, w2_ref, b2_ref, w3_ref, b3_ref,
                     e_ref, o_ref):
    i = pl.program_id(0)

    @pl.when(i == 0)
    def _():
        o_ref[...] = jnp.zeros_like(o_ref)

    m = _scale_map(d0_ref[...], d1_ref[...], e_ref[...])
    u = (u0_ref[...] + u1_ref[...]) * m
    h = jnp.maximum(jnp.dot(u, w2b_ref[...], preferred_element_type=_f32)
                    + b2b_ref[...], 0.0)
    l1 = jnp.maximum(jnp.dot(h, w1_ref[...], preferred_element_type=_f32)
                     + b1_ref[...], 0.0)
    l2 = jnp.maximum(jnp.dot(l1, w2_ref[...], preferred_element_type=_f32)
                     + b2_ref[...], 0.0)
    lo = jnp.dot(l2, w3_ref[...], preferred_element_type=_f32) + b3_ref[...]
    r_ids = (8 * (BLR * i + lax.broadcasted_iota(jnp.int32, (BLR, 8), 0))
             + lax.broadcasted_iota(jnp.int32, (BLR, 8), 1))
    lo = jnp.where(r_ids < N_VAR, lo, 0.0)
    o_ref[...] = o_ref[...] + jnp.sum(lo) * (1.0 / N_VAR)


def _tc_readout(u_p, dv_flat, w2b, b2b, wo1, bo1, wo2, bo2, wo3, bo3):
    eye8 = jnp.eye(8, dtype=_f32)
    u2 = u_p.reshape(2 * NV8, 128)
    dv2 = dv_flat.reshape(2 * NV8, 8)
    return pl.pallas_call(
        _tc_readout_body,
        grid=(2,),
        in_specs=[
            pl.BlockSpec((BLR, 128), lambda i: (i, 0)),
            pl.BlockSpec((BLR, 128), lambda i: (2 + i, 0)),
            pl.BlockSpec((BLR, 8), lambda i: (i, 0)),
            pl.BlockSpec((BLR, 8), lambda i: (2 + i, 0)),
            pl.BlockSpec((128, 128), lambda i: (0, 0)),
            pl.BlockSpec((1, 128), lambda i: (0, 0)),
            pl.BlockSpec((128, 128), lambda i: (0, 0)),
            pl.BlockSpec((1, 128), lambda i: (0, 0)),
            pl.BlockSpec((128, 128), lambda i: (0, 0)),
            pl.BlockSpec((1, 128), lambda i: (0, 0)),
            pl.BlockSpec((128, 8), lambda i: (0, 0)),
            pl.BlockSpec((1, 8), lambda i: (0, 0)),
            pl.BlockSpec((8, 128), lambda i: (0, 0)),
        ],
        out_specs=pl.BlockSpec((1, 1), lambda i: (0, 0)),
        out_shape=jax.ShapeDtypeStruct((1, 1), _f32),
    )(u2, u2, dv2, dv2,
      jnp.kron(eye8, w2b), jnp.tile(b2b, 8).reshape(1, 128),
      jnp.kron(eye8, wo1), jnp.tile(bo1, 8).reshape(1, 128),
      jnp.kron(eye8, wo2), jnp.tile(bo2, 8).reshape(1, 128),
      jnp.kron(eye8, wo3), jnp.tile(bo3, 8).reshape(1, 8), _emat())


def kernel(x_var, x_con, x_soc, v2c_src, v2c_dst, s2c_src, s2c_dst,
           Wv, bv, Wc, bc, Ws, bs,
           W1f, b1f, W1b, b1b, W2f, b2f, W2b, b2b,
           Wo1, bo1, Wo2, bo2, Wo3, bo3):
    del x_con, Wc, bc, W1f, b1f, W1b, b1b
    zeros1 = jnp.zeros((NVP,), _f32)
    zeros2 = jnp.zeros((KS, H), _f32)
    onesv = jnp.ones((KD,), _f32)
    xv128 = x_var.reshape(NVR8, 72)
    xs128 = x_soc.reshape(NSR8, 8)

    tv0 = _tc_embed_var(xv128, Wv, bv)
    dv_p, ds_p = _sc_deg_src(v2c_src, s2c_src, zeros1, onesv)
    tv = _tc_scale_var(tv0, dv_p).reshape(N_VAR, H)
    ts = _tc_table_soc(xs128, ds_p, Ws, bs).reshape(N_SOC, H)
    sv_p, dcv_p = _sc_fwd_v(v2c_src, v2c_dst, tv, zeros2, zeros1, onesv)
    ss_p, dcs_p = _sc_fwd_s(s2c_src, s2c_dst, ts, zeros2, zeros1, onesv)
    tc = _tc_con(sv_p, ss_p, dcv_p, dcs_p, W2f, b2f).reshape(NCP, H)
    u_p = _sc_bwd(v2c_dst, v2c_src, tc, zeros2)
    out = _tc_readout(u_p, dv_p, W2b, b2b, Wo1, bo1, Wo2, bo2, Wo3, bo3)
    return out
